# baseline (speedup 1.0000x reference)
import functools

import jax
import jax.numpy as jnp
from jax import lax
from jax.experimental import pallas as pl
from jax.experimental.pallas import tpu as pltpu
from jax.experimental.pallas import tpu_sc as plsc

NC = 2
NS = 16
MB = 128

_MESH = plsc.VectorSubcoreMesh(
    core_axis_name="c", subcore_axis_name="s", num_cores=NC, num_subcores=NS)


def _sc_degrees(edge_index, zpad, ones128, NP):
    E = edge_index.shape[1]
    nmb = E // MB
    nmb_core = nmb // NC
    rows_t = NP // NS

    @functools.partial(
        pl.kernel,
        out_type=jax.ShapeDtypeStruct((NC, 2, NP, 128), jnp.float32),
        mesh=_MESH,
        scratch_types=[
            pltpu.VMEM((MB,), jnp.int32),
            pltpu.VMEM((MB, 128), jnp.float32),
            pltpu.VMEM_SHARED((NP, 128), jnp.float32),
        ],
    )
    def deg_kernel(ei, zp, ones_hbm, out, idx_v, ones_v, acc):
        c = lax.axis_index("c")
        s = lax.axis_index("s")
        r0 = s * rows_t
        pltpu.sync_copy(ones_hbm, ones_v)
        nit = nmb_core // NS

        def run_pass(which):
            pltpu.sync_copy(zp.at[pl.ds(r0, rows_t), :],
                            acc.at[pl.ds(r0, rows_t), :])
            plsc.subcore_barrier()

            def body(i, _):
                m = c * nmb_core + i * NS + s
                base = m * MB
                pltpu.sync_copy(ei.at[which, pl.ds(base, MB)], idx_v)
                pltpu.sync_copy(ones_v, acc.at[idx_v], add=True)
                return 0

            lax.fori_loop(0, nit, body, 0)
            plsc.subcore_barrier()
            pltpu.sync_copy(acc.at[pl.ds(r0, rows_t), :],
                            out.at[c, which, pl.ds(r0, rows_t), :])

        run_pass(0)
        plsc.subcore_barrier()
        run_pass(1)

    return deg_kernel(edge_index, zpad, ones128)


def _agg_edge_loop(tab, ei, acc, sidx2, didx2, rows0, rows1, sem0, sem1,
                   mb0_of, nit):
    def load_sidx(i, buf):
        pltpu.sync_copy(ei.at[0, pl.ds(mb0_of(i) * MB, MB)],
                        sidx2.at[buf])

    def load_didx(i, buf):
        pltpu.sync_copy(ei.at[1, pl.ds(mb0_of(i) * MB, MB)],
                        didx2.at[buf])

    load_sidx(0, 0)
    pltpu.async_copy(tab.at[sidx2.at[0]], rows0, sem0)

    def body(g, _):
        load_sidx(2 * g + 1, 1)
        pltpu.async_copy(tab.at[sidx2.at[1]], rows1, sem1)
        pltpu.make_async_copy(tab.at[sidx2.at[0]], rows0, sem0).wait()
        load_didx(2 * g, 0)
        pltpu.sync_copy(rows0, acc.at[didx2.at[0]], add=True)

        @pl.when(2 * g + 2 < nit)
        def _():
            load_sidx(2 * g + 2, 0)
            pltpu.async_copy(tab.at[sidx2.at[0]], rows0, sem0)

        pltpu.make_async_copy(tab.at[sidx2.at[1]], rows1, sem1).wait()
        load_didx(2 * g + 1, 1)
        pltpu.sync_copy(rows1, acc.at[didx2.at[1]], add=True)
        return 0

    lax.fori_loop(0, nit // 2, body, 0)


def _sc_agg128(table, edge_index, zpad, NP):
    E = edge_index.shape[1]
    nmb = E // MB
    nmb_core = nmb // NC
    rows_t = NP // NS
    nit = nmb_core // NS

    @functools.partial(
        pl.kernel,
        out_type=jax.ShapeDtypeStruct((NC, NP, 128), jnp.float32),
        mesh=_MESH,
        scratch_types=[
            pltpu.VMEM((2, MB), jnp.int32),
            pltpu.VMEM((2, MB), jnp.int32),
            pltpu.VMEM((MB, 128), jnp.float32),
            pltpu.VMEM((MB, 128), jnp.float32),
            pltpu.VMEM_SHARED((NP, 128), jnp.float32),
            pltpu.SemaphoreType.DMA,
            pltpu.SemaphoreType.DMA,
        ],
    )
    def agg_kernel(tab, ei, zp, out, sidx2, didx2, rows0, rows1, acc,
                   sem0, sem1):
        c = lax.axis_index("c")
        s = lax.axis_index("s")
        r0 = s * rows_t
        pltpu.sync_copy(zp.at[pl.ds(r0, rows_t), :],
                        acc.at[pl.ds(r0, rows_t), :])
        plsc.subcore_barrier()
        _agg_edge_loop(tab, ei, acc, sidx2, didx2, rows0, rows1, sem0, sem1,
                       lambda i: c * nmb_core + i * NS + s, nit)
        plsc.subcore_barrier()
        pltpu.sync_copy(acc.at[pl.ds(r0, rows_t), :],
                        out.at[c, pl.ds(r0, rows_t), :])

    return agg_kernel(table, edge_index, zpad)


def _sc_agg512(t0, t1, t2, t3, edge_index, zpad, NP):
    E = edge_index.shape[1]
    nmb = E // MB
    rows_t = NP // NS
    ot = jax.ShapeDtypeStruct((NP, 128), jnp.float32)

    nit = nmb // NS

    @functools.partial(
        pl.kernel,
        out_type=(ot, ot, ot, ot),
        mesh=_MESH,
        scratch_types=[
            pltpu.VMEM((2, MB), jnp.int32),
            pltpu.VMEM((2, MB), jnp.int32),
            pltpu.VMEM((MB, 128), jnp.float32),
            pltpu.VMEM((MB, 128), jnp.float32),
            pltpu.VMEM_SHARED((NP, 128), jnp.float32),
            pltpu.SemaphoreType.DMA,
            pltpu.SemaphoreType.DMA,
        ],
    )
    def agg_kernel(a0, a1, a2, a3, ei, zp, o0, o1, o2, o3,
                   sidx2, didx2, rows0, rows1, acc, sem0, sem1):
        c = lax.axis_index("c")
        s = lax.axis_index("s")
        r0 = s * rows_t

        def run_chunk(tab, out):
            pltpu.sync_copy(zp.at[pl.ds(r0, rows_t), :],
                            acc.at[pl.ds(r0, rows_t), :])
            plsc.subcore_barrier()
            _agg_edge_loop(tab, ei, acc, sidx2, didx2, rows0, rows1,
                           sem0, sem1, lambda i: i * NS + s, nit)
            plsc.subcore_barrier()
            pltpu.sync_copy(acc.at[pl.ds(r0, rows_t), :],
                            out.at[pl.ds(r0, rows_t), :])

        @pl.when(c == 0)
        def _():
            run_chunk(a0, o0)
            plsc.subcore_barrier()
            run_chunk(a1, o1)

        @pl.when(c == 1)
        def _():
            run_chunk(a2, o2)
            plsc.subcore_barrier()
            run_chunk(a3, o3)

    return agg_kernel(t0, t1, t2, t3, edge_index, zpad)


def _tc_prep(x_pad, degp, maskcol, token, NP, MBK):
    grid = NP // MBK

    def body(x_ref, deg_ref, m_ref, tok_ref, oxn_ref, ni_ref, no_ref, mns_ref):
        dego = deg_ref[0, 0, :, 0:1] + deg_ref[1, 0, :, 0:1]
        degi = deg_ref[0, 1, :, 0:1] + deg_ref[1, 1, :, 0:1]
        no = jnp.where(dego > 0, lax.rsqrt(jnp.maximum(dego, 1e-30)), 0.0)
        ni = jnp.where(degi > 0, lax.rsqrt(jnp.maximum(degi, 1e-30)), 0.0)
        m = m_ref[...]
        ox = x_ref[...] * m + (1.0 - m) * tok_ref[...]
        oxn_ref[...] = ox * no
        ni_ref[...] = ni
        no_ref[...] = no
        mns_ref[...] = m * no

    return pl.pallas_call(
        body,
        grid=(grid,),
        in_specs=[
            pl.BlockSpec((MBK, 128), lambda i: (i, 0)),
            pl.BlockSpec((2, 2, MBK, 128), lambda i: (0, 0, i, 0)),
            pl.BlockSpec((MBK, 1), lambda i: (i, 0)),
            pl.BlockSpec((1, 128), lambda i: (0, 0)),
        ],
        out_specs=[
            pl.BlockSpec((MBK, 128), lambda i: (i, 0)),
            pl.BlockSpec((MBK, 1), lambda i: (i, 0)),
            pl.BlockSpec((MBK, 1), lambda i: (i, 0)),
            pl.BlockSpec((MBK, 1), lambda i: (i, 0)),
        ],
        out_shape=[
            jax.ShapeDtypeStruct((NP, 128), jnp.float32),
            jax.ShapeDtypeStruct((NP, 1), jnp.float32),
            jax.ShapeDtypeStruct((NP, 1), jnp.float32),
            jax.ShapeDtypeStruct((NP, 1), jnp.float32),
        ],
    )(x_pad, degp, maskcol, token)


def _tc_wed(W_e2d, W_dec):
    def body(a_ref, b_ref, o_ref):
        o_ref[...] = jnp.dot(a_ref[...], b_ref[...],
                             preferred_element_type=jnp.float32)

    return pl.pallas_call(
        body,
        out_shape=jax.ShapeDtypeStruct((512, 128), jnp.float32),
    )(W_e2d, W_dec)


def _tc_layer1(agg1, W1, b1, normin, normout, NP, MBK):
    grid = (NP // MBK, 4)

    def body(a_ref, w_ref, b_ref, ni_ref, no_ref, o_ref):
        a = a_ref[0] + a_ref[1]
        acc = jnp.dot(a, w_ref[...], preferred_element_type=jnp.float32)
        h = jnp.maximum(acc * ni_ref[...] + b_ref[...], 0.0)
        o_ref[0] = h * no_ref[...]

    return pl.pallas_call(
        body,
        grid=grid,
        in_specs=[
            pl.BlockSpec((2, MBK, 128), lambda i, c: (0, i, 0)),
            pl.BlockSpec((128, 128), lambda i, c: (0, c)),
            pl.BlockSpec((1, 128), lambda i, c: (0, c)),
            pl.BlockSpec((MBK, 1), lambda i, c: (i, 0)),
            pl.BlockSpec((MBK, 1), lambda i, c: (i, 0)),
        ],
        out_specs=pl.BlockSpec((1, MBK, 128), lambda i, c: (c, i, 0)),
        out_shape=jax.ShapeDtypeStruct((4, NP, 128), jnp.float32),
    )(agg1, W1, b1, normin, normout)


def _tc_layer2(agg2, W2, b2, normin, mns, W_ed, NP, MBK):
    grid = (NP // MBK,)

    def body(a_ref, w_ref, b_ref, ni_ref, mns_ref, wed_ref, enc_ref, d_ref):
        acc = jnp.dot(a_ref[0], w_ref[pl.ds(0, 128), :],
                      preferred_element_type=jnp.float32)
        for cc in range(1, 4):
            acc += jnp.dot(a_ref[cc], w_ref[pl.ds(cc * 128, 128), :],
                           preferred_element_type=jnp.float32)
        enc = jnp.maximum(acc * ni_ref[...] + b_ref[...], 0.0)
        enc_ref[...] = enc
        d_ref[...] = jnp.dot(enc * mns_ref[...], wed_ref[...],
                             preferred_element_type=jnp.float32)

    return pl.pallas_call(
        body,
        grid=grid,
        in_specs=[
            pl.BlockSpec((4, MBK, 128), lambda i: (0, i, 0)),
            pl.BlockSpec((512, 512), lambda i: (0, 0)),
            pl.BlockSpec((1, 512), lambda i: (0, 0)),
            pl.BlockSpec((MBK, 1), lambda i: (i, 0)),
            pl.BlockSpec((MBK, 1), lambda i: (i, 0)),
            pl.BlockSpec((512, 128), lambda i: (0, 0)),
        ],
        out_specs=[
            pl.BlockSpec((MBK, 512), lambda i: (i, 0)),
            pl.BlockSpec((MBK, 128), lambda i: (i, 0)),
        ],
        out_shape=[
            jax.ShapeDtypeStruct((NP, 512), jnp.float32),
            jax.ShapeDtypeStruct((NP, 128), jnp.float32),
        ],
    )(agg2, W2, b2, normin, mns, W_ed)


def _tc_final(agg3, b_dec, normin, maskcol, x_pad, NP, MBK):
    grid = (NP // MBK,)

    def body(a_ref, b_ref, ni_ref, m_ref, x_ref, rec_ref, loss_ref):
        i = pl.program_id(0)
        r = (a_ref[0] + a_ref[1]) * ni_ref[...] + b_ref[...]
        rec_ref[...] = r
        w = 1.0 - m_ref[...]
        x = x_ref[...]
        rnorm = jnp.sqrt(jnp.sum(r * r, axis=-1, keepdims=True))
        xnorm = jnp.sqrt(jnp.sum(x * x, axis=-1, keepdims=True))
        rn = r / jnp.maximum(rnorm, 1e-12)
        xn = x / jnp.maximum(xnorm, 1e-12)
        cos = jnp.sum(rn * xn, axis=-1, keepdims=True)
        contrib = jnp.sum(w * (1.0 - cos) ** 2, keepdims=True).reshape(1, 1)

        @pl.when(i == 0)
        def _():
            loss_ref[...] = contrib

        @pl.when(i > 0)
        def _():
            loss_ref[...] += contrib

    return pl.pallas_call(
        body,
        grid=grid,
        in_specs=[
            pl.BlockSpec((2, MBK, 128), lambda i: (0, i, 0)),
            pl.BlockSpec((1, 128), lambda i: (0, 0)),
            pl.BlockSpec((MBK, 1), lambda i: (i, 0)),
            pl.BlockSpec((MBK, 1), lambda i: (i, 0)),
            pl.BlockSpec((MBK, 128), lambda i: (i, 0)),
        ],
        out_specs=[
            pl.BlockSpec((MBK, 128), lambda i: (i, 0)),
            pl.BlockSpec((1, 1), lambda i: (0, 0)),
        ],
        out_shape=[
            jax.ShapeDtypeStruct((NP, 128), jnp.float32),
            jax.ShapeDtypeStruct((1, 1), jnp.float32),
        ],
    )(agg3, b_dec, normin, maskcol, x_pad)


def kernel(x, edge_index, mask_nodes, enc_mask_token,
           W_enc1, b_enc1, W_enc2, b_enc2, W_e2d, W_dec, b_dec):
    N = x.shape[0]
    num_mask = mask_nodes.shape[0]
    NP = ((N + NS * 40 - 1) // (NS * 40)) * (NS * 40)
    MBK = NP // 8

    x_pad = jnp.pad(x, ((0, NP - N), (0, 0)))
    maskcol = jnp.ones((NP, 1), jnp.float32).at[mask_nodes].set(0.0)
    zpad = jnp.zeros((NP, 128), jnp.float32)
    ones128 = jnp.ones((MB, 128), jnp.float32)

    E = edge_index.shape[1]
    EDIV = MB * NS * NC * 2
    EP = ((E + EDIV - 1) // EDIV) * EDIV
    edge_index = jnp.concatenate(
        [edge_index, jnp.full((2, EP - E), NP - 1, jnp.int32)], axis=1)

    degp = _sc_degrees(edge_index, zpad, ones128, NP)
    oxn, normin, normout, mns = _tc_prep(
        x_pad, degp, maskcol, enc_mask_token, NP, MBK)
    W_ed = _tc_wed(W_e2d, W_dec)

    agg1 = _sc_agg128(oxn, edge_index, zpad, NP)
    h1n = _tc_layer1(agg1, W_enc1, b_enc1.reshape(1, -1), normin, normout,
                     NP, MBK)
    agg2c = _sc_agg512(h1n[0], h1n[1], h1n[2], h1n[3], edge_index, zpad, NP)
    agg2 = jnp.stack(agg2c)
    enc_pad, d = _tc_layer2(agg2, W_enc2, b_enc2.reshape(1, -1), normin, mns,
                            W_ed, NP, MBK)
    agg3 = _sc_agg128(d, edge_index, zpad, NP)
    recon_pad, loss_acc = _tc_final(agg3, b_dec.reshape(1, -1), normin,
                                    maskcol, x_pad, NP, MBK)

    enc_rep = enc_pad[:N]
    recon = recon_pad[:N]
    loss = (loss_acc[0, 0] / num_mask).astype(jnp.float32)
    return (enc_rep, recon, loss)


# speedup vs baseline: 1.9132x; 1.9132x over previous
import functools

import jax
import jax.numpy as jnp
from jax import lax
from jax.experimental import pallas as pl
from jax.experimental.pallas import tpu as pltpu
from jax.experimental.pallas import tpu_sc as plsc

NC = 2
NS = 16
MB = 128

_MESH = plsc.VectorSubcoreMesh(
    core_axis_name="c", subcore_axis_name="s", num_cores=NC, num_subcores=NS)


def _sc_degrees(edge_index, zpad, ones128, NP):
    E = edge_index.shape[1]
    nmb = E // MB
    nmb_core = nmb // NC
    rows_t = NP // NS

    @functools.partial(
        pl.kernel,
        out_type=jax.ShapeDtypeStruct((NC, 2, NP, 128), jnp.float32),
        mesh=_MESH,
        scratch_types=[
            pltpu.VMEM((MB,), jnp.int32),
            pltpu.VMEM((MB, 128), jnp.float32),
            pltpu.VMEM_SHARED((NP, 128), jnp.float32),
        ],
    )
    def deg_kernel(ei, zp, ones_hbm, out, idx_v, ones_v, acc):
        c = lax.axis_index("c")
        s = lax.axis_index("s")
        r0 = s * rows_t
        pltpu.sync_copy(ones_hbm, ones_v)
        nit = nmb_core // NS

        def run_pass(which):
            pltpu.sync_copy(zp.at[pl.ds(r0, rows_t), :],
                            acc.at[pl.ds(r0, rows_t), :])
            plsc.subcore_barrier()

            def body(i, _):
                m = c * nmb_core + i * NS + s
                base = m * MB
                pltpu.sync_copy(ei.at[which, pl.ds(base, MB)], idx_v)
                pltpu.sync_copy(ones_v, acc.at[idx_v], add=True)
                return 0

            lax.fori_loop(0, nit, body, 0)
            plsc.subcore_barrier()
            pltpu.sync_copy(acc.at[pl.ds(r0, rows_t), :],
                            out.at[c, which, pl.ds(r0, rows_t), :])

        run_pass(0)
        plsc.subcore_barrier()
        run_pass(1)

    return deg_kernel(edge_index, zpad, ones128)


def _agg_edge_loop(tab, ei, acc, sidx2, didx2, rows0, rows1, sem0, sem1,
                   mb0_of, nit):
    def load_sidx(i, buf):
        pltpu.sync_copy(ei.at[0, pl.ds(mb0_of(i) * MB, MB)],
                        sidx2.at[buf])

    def load_didx(i, buf):
        pltpu.sync_copy(ei.at[1, pl.ds(mb0_of(i) * MB, MB)],
                        didx2.at[buf])

    load_sidx(0, 0)
    pltpu.async_copy(tab.at[sidx2.at[0]], rows0, sem0)

    def body(g, _):
        load_sidx(2 * g + 1, 1)
        pltpu.async_copy(tab.at[sidx2.at[1]], rows1, sem1)
        pltpu.make_async_copy(tab.at[sidx2.at[0]], rows0, sem0).wait()
        load_didx(2 * g, 0)
        pltpu.sync_copy(rows0, acc.at[didx2.at[0]], add=True)

        @pl.when(2 * g + 2 < nit)
        def _():
            load_sidx(2 * g + 2, 0)
            pltpu.async_copy(tab.at[sidx2.at[0]], rows0, sem0)

        pltpu.make_async_copy(tab.at[sidx2.at[1]], rows1, sem1).wait()
        load_didx(2 * g + 1, 1)
        pltpu.sync_copy(rows1, acc.at[didx2.at[1]], add=True)
        return 0

    lax.fori_loop(0, nit // 2, body, 0)


def _sc_agg128(table, edge_index, zpad, NP):
    E = edge_index.shape[1]
    nmb = E // MB
    nmb_core = nmb // NC
    rows_t = NP // NS
    nit = nmb_core // NS

    @functools.partial(
        pl.kernel,
        out_type=jax.ShapeDtypeStruct((NC, NP, 128), jnp.float32),
        mesh=_MESH,
        scratch_types=[
            pltpu.VMEM((2, MB), jnp.int32),
            pltpu.VMEM((2, MB), jnp.int32),
            pltpu.VMEM((MB, 128), jnp.float32),
            pltpu.VMEM((MB, 128), jnp.float32),
            pltpu.VMEM_SHARED((NP, 128), jnp.float32),
            pltpu.SemaphoreType.DMA,
            pltpu.SemaphoreType.DMA,
        ],
    )
    def agg_kernel(tab, ei, zp, out, sidx2, didx2, rows0, rows1, acc,
                   sem0, sem1):
        c = lax.axis_index("c")
        s = lax.axis_index("s")
        r0 = s * rows_t
        pltpu.sync_copy(zp.at[pl.ds(r0, rows_t), :],
                        acc.at[pl.ds(r0, rows_t), :])
        plsc.subcore_barrier()
        _agg_edge_loop(tab, ei, acc, sidx2, didx2, rows0, rows1, sem0, sem1,
                       lambda i: c * nmb_core + i * NS + s, nit)
        plsc.subcore_barrier()
        pltpu.sync_copy(acc.at[pl.ds(r0, rows_t), :],
                        out.at[c, pl.ds(r0, rows_t), :])

    return agg_kernel(table, edge_index, zpad)


def _sc_agg512(t0, t1, t2, t3, edge_index, zpad, NP):
    E = edge_index.shape[1]
    nmb = E // MB
    rows_t = NP // NS
    ot = jax.ShapeDtypeStruct((NP, 128), jnp.float32)

    nit = nmb // NS

    @functools.partial(
        pl.kernel,
        out_type=(ot, ot, ot, ot),
        mesh=_MESH,
        scratch_types=[
            pltpu.VMEM((2, MB), jnp.int32),
            pltpu.VMEM((2, MB), jnp.int32),
            pltpu.VMEM((MB, 128), jnp.float32),
            pltpu.VMEM((MB, 128), jnp.float32),
            pltpu.VMEM_SHARED((NP, 128), jnp.float32),
            pltpu.SemaphoreType.DMA,
            pltpu.SemaphoreType.DMA,
        ],
    )
    def agg_kernel(a0, a1, a2, a3, ei, zp, o0, o1, o2, o3,
                   sidx2, didx2, rows0, rows1, acc, sem0, sem1):
        c = lax.axis_index("c")
        s = lax.axis_index("s")
        r0 = s * rows_t

        def run_chunk(tab, out):
            pltpu.sync_copy(zp.at[pl.ds(r0, rows_t), :],
                            acc.at[pl.ds(r0, rows_t), :])
            plsc.subcore_barrier()
            _agg_edge_loop(tab, ei, acc, sidx2, didx2, rows0, rows1,
                           sem0, sem1, lambda i: i * NS + s, nit)
            plsc.subcore_barrier()
            pltpu.sync_copy(acc.at[pl.ds(r0, rows_t), :],
                            out.at[pl.ds(r0, rows_t), :])

        @pl.when(c == 0)
        def _():
            run_chunk(a0, o0)
            plsc.subcore_barrier()
            run_chunk(a1, o1)

        @pl.when(c == 1)
        def _():
            run_chunk(a2, o2)
            plsc.subcore_barrier()
            run_chunk(a3, o3)

    return agg_kernel(t0, t1, t2, t3, edge_index, zpad)


def _tc_prep(x_pad, degp, maskcol, token, NP, MBK):
    grid = NP // MBK

    def body(x_ref, deg_ref, m_ref, tok_ref, oxn_ref, ni_ref, no_ref, mns_ref):
        dego = deg_ref[0, 0, :, 0:1] + deg_ref[1, 0, :, 0:1]
        degi = deg_ref[0, 1, :, 0:1] + deg_ref[1, 1, :, 0:1]
        no = jnp.where(dego > 0, lax.rsqrt(jnp.maximum(dego, 1e-30)), 0.0)
        ni = jnp.where(degi > 0, lax.rsqrt(jnp.maximum(degi, 1e-30)), 0.0)
        m = m_ref[...]
        ox = x_ref[...] * m + (1.0 - m) * tok_ref[...]
        oxn_ref[...] = ox * no
        ni_ref[...] = ni
        no_ref[...] = no
        mns_ref[...] = m * no

    return pl.pallas_call(
        body,
        grid=(grid,),
        in_specs=[
            pl.BlockSpec((MBK, 128), lambda i: (i, 0)),
            pl.BlockSpec((2, 2, MBK, 128), lambda i: (0, 0, i, 0)),
            pl.BlockSpec((MBK, 1), lambda i: (i, 0)),
            pl.BlockSpec((1, 128), lambda i: (0, 0)),
        ],
        out_specs=[
            pl.BlockSpec((MBK, 128), lambda i: (i, 0)),
            pl.BlockSpec((MBK, 1), lambda i: (i, 0)),
            pl.BlockSpec((MBK, 1), lambda i: (i, 0)),
            pl.BlockSpec((MBK, 1), lambda i: (i, 0)),
        ],
        out_shape=[
            jax.ShapeDtypeStruct((NP, 128), jnp.float32),
            jax.ShapeDtypeStruct((NP, 1), jnp.float32),
            jax.ShapeDtypeStruct((NP, 1), jnp.float32),
            jax.ShapeDtypeStruct((NP, 1), jnp.float32),
        ],
    )(x_pad, degp, maskcol, token)


def _tc_wed(W_e2d, W_dec):
    def body(a_ref, b_ref, o_ref):
        o_ref[...] = jnp.dot(a_ref[...], b_ref[...],
                             preferred_element_type=jnp.float32)

    return pl.pallas_call(
        body,
        out_shape=jax.ShapeDtypeStruct((512, 128), jnp.float32),
    )(W_e2d, W_dec)


def _tc_layer1(agg1, W1, b1, normin, normout, NP, MBK):
    grid = (NP // MBK, 4)

    def body(a_ref, w_ref, b_ref, ni_ref, no_ref, o_ref):
        a = a_ref[0] + a_ref[1]
        acc = jnp.dot(a, w_ref[...], preferred_element_type=jnp.float32)
        h = jnp.maximum(acc * ni_ref[...] + b_ref[...], 0.0)
        o_ref[0] = h * no_ref[...]

    return pl.pallas_call(
        body,
        grid=grid,
        in_specs=[
            pl.BlockSpec((2, MBK, 128), lambda i, c: (0, i, 0)),
            pl.BlockSpec((128, 128), lambda i, c: (0, c)),
            pl.BlockSpec((1, 128), lambda i, c: (0, c)),
            pl.BlockSpec((MBK, 1), lambda i, c: (i, 0)),
            pl.BlockSpec((MBK, 1), lambda i, c: (i, 0)),
        ],
        out_specs=pl.BlockSpec((1, MBK, 128), lambda i, c: (c, i, 0)),
        out_shape=jax.ShapeDtypeStruct((4, NP, 128), jnp.float32),
    )(agg1, W1, b1, normin, normout)


def _tc_layer2(agg2, W2, b2, normin, mns, W_ed, NP, MBK):
    grid = (NP // MBK,)

    def body(a_ref, w_ref, b_ref, ni_ref, mns_ref, wed_ref, enc_ref, d_ref):
        acc = jnp.dot(a_ref[0], w_ref[pl.ds(0, 128), :],
                      preferred_element_type=jnp.float32)
        for cc in range(1, 4):
            acc += jnp.dot(a_ref[cc], w_ref[pl.ds(cc * 128, 128), :],
                           preferred_element_type=jnp.float32)
        enc = jnp.maximum(acc * ni_ref[...] + b_ref[...], 0.0)
        enc_ref[...] = enc
        d_ref[...] = jnp.dot(enc * mns_ref[...], wed_ref[...],
                             preferred_element_type=jnp.float32)

    return pl.pallas_call(
        body,
        grid=grid,
        in_specs=[
            pl.BlockSpec((4, MBK, 128), lambda i: (0, i, 0)),
            pl.BlockSpec((512, 512), lambda i: (0, 0)),
            pl.BlockSpec((1, 512), lambda i: (0, 0)),
            pl.BlockSpec((MBK, 1), lambda i: (i, 0)),
            pl.BlockSpec((MBK, 1), lambda i: (i, 0)),
            pl.BlockSpec((512, 128), lambda i: (0, 0)),
        ],
        out_specs=[
            pl.BlockSpec((MBK, 512), lambda i: (i, 0)),
            pl.BlockSpec((MBK, 128), lambda i: (i, 0)),
        ],
        out_shape=[
            jax.ShapeDtypeStruct((NP, 512), jnp.float32),
            jax.ShapeDtypeStruct((NP, 128), jnp.float32),
        ],
    )(agg2, W2, b2, normin, mns, W_ed)


def _tc_final(agg3, b_dec, normin, maskcol, x_pad, NP, MBK):
    grid = (NP // MBK,)

    def body(a_ref, b_ref, ni_ref, m_ref, x_ref, rec_ref, loss_ref):
        i = pl.program_id(0)
        r = (a_ref[0] + a_ref[1]) * ni_ref[...] + b_ref[...]
        rec_ref[...] = r
        w = 1.0 - m_ref[...]
        x = x_ref[...]
        rnorm = jnp.sqrt(jnp.sum(r * r, axis=-1, keepdims=True))
        xnorm = jnp.sqrt(jnp.sum(x * x, axis=-1, keepdims=True))
        rn = r / jnp.maximum(rnorm, 1e-12)
        xn = x / jnp.maximum(xnorm, 1e-12)
        cos = jnp.sum(rn * xn, axis=-1, keepdims=True)
        contrib = jnp.sum(w * (1.0 - cos) ** 2, keepdims=True).reshape(1, 1)

        @pl.when(i == 0)
        def _():
            loss_ref[...] = contrib

        @pl.when(i > 0)
        def _():
            loss_ref[...] += contrib

    return pl.pallas_call(
        body,
        grid=grid,
        in_specs=[
            pl.BlockSpec((2, MBK, 128), lambda i: (0, i, 0)),
            pl.BlockSpec((1, 128), lambda i: (0, 0)),
            pl.BlockSpec((MBK, 1), lambda i: (i, 0)),
            pl.BlockSpec((MBK, 1), lambda i: (i, 0)),
            pl.BlockSpec((MBK, 128), lambda i: (i, 0)),
        ],
        out_specs=[
            pl.BlockSpec((MBK, 128), lambda i: (i, 0)),
            pl.BlockSpec((1, 1), lambda i: (0, 0)),
        ],
        out_shape=[
            jax.ShapeDtypeStruct((NP, 128), jnp.float32),
            jax.ShapeDtypeStruct((1, 1), jnp.float32),
        ],
    )(agg3, b_dec, normin, maskcol, x_pad)


def kernel(x, edge_index, mask_nodes, enc_mask_token,
           W_enc1, b_enc1, W_enc2, b_enc2, W_e2d, W_dec, b_dec):
    N = x.shape[0]
    num_mask = mask_nodes.shape[0]
    NP = ((N + NS * 40 - 1) // (NS * 40)) * (NS * 40)
    MBK = NP // 8

    x_pad = jnp.pad(x, ((0, NP - N), (0, 0)))
    maskcol = jnp.ones((NP, 1), jnp.float32).at[mask_nodes].set(0.0)
    zpad = jnp.zeros((NP, 128), jnp.float32)
    ones128 = jnp.ones((MB, 128), jnp.float32)

    E = edge_index.shape[1]
    EDIV = MB * NS * NC * 2
    EP = ((E + EDIV - 1) // EDIV) * EDIV
    pad_nodes = N + jnp.arange(EP - E, dtype=jnp.int32) % (NP - N)
    edge_index = jnp.concatenate(
        [edge_index, jnp.stack([pad_nodes, pad_nodes])], axis=1)

    degp = _sc_degrees(edge_index, zpad, ones128, NP)
    oxn, normin, normout, mns = _tc_prep(
        x_pad, degp, maskcol, enc_mask_token, NP, MBK)
    W_ed = _tc_wed(W_e2d, W_dec)

    agg1 = _sc_agg128(oxn, edge_index, zpad, NP)
    h1n = _tc_layer1(agg1, W_enc1, b_enc1.reshape(1, -1), normin, normout,
                     NP, MBK)
    agg2c = _sc_agg512(h1n[0], h1n[1], h1n[2], h1n[3], edge_index, zpad, NP)
    agg2 = jnp.stack(agg2c)
    enc_pad, d = _tc_layer2(agg2, W_enc2, b_enc2.reshape(1, -1), normin, mns,
                            W_ed, NP, MBK)
    agg3 = _sc_agg128(d, edge_index, zpad, NP)
    recon_pad, loss_acc = _tc_final(agg3, b_dec.reshape(1, -1), normin,
                                    maskcol, x_pad, NP, MBK)

    enc_rep = enc_pad[:N]
    recon = recon_pad[:N]
    loss = (loss_acc[0, 0] / num_mask).astype(jnp.float32)
    return (enc_rep, recon, loss)


# speedup vs baseline: 2.1720x; 1.1353x over previous
import functools

import jax
import jax.numpy as jnp
from jax import lax
from jax.experimental import pallas as pl
from jax.experimental.pallas import tpu as pltpu
from jax.experimental.pallas import tpu_sc as plsc

NC = 2
NS = 16
MB = 128

_MESH = plsc.VectorSubcoreMesh(
    core_axis_name="c", subcore_axis_name="s", num_cores=NC, num_subcores=NS)


def _sc_degrees(edge_index, zpad, ones128, NP):
    E = edge_index.shape[1]
    nmb = E // MB
    nmb_core = nmb // NC
    rows_t = NP // NS

    @functools.partial(
        pl.kernel,
        out_type=jax.ShapeDtypeStruct((NC, 2, NP, 128), jnp.float32),
        mesh=_MESH,
        scratch_types=[
            pltpu.VMEM((MB,), jnp.int32),
            pltpu.VMEM((MB, 128), jnp.float32),
            pltpu.VMEM_SHARED((NP, 128), jnp.float32),
        ],
    )
    def deg_kernel(ei, zp, ones_hbm, out, idx_v, ones_v, acc):
        c = lax.axis_index("c")
        s = lax.axis_index("s")
        r0 = s * rows_t
        pltpu.sync_copy(ones_hbm, ones_v)
        nit = nmb_core // NS

        def run_pass(which):
            pltpu.sync_copy(zp.at[pl.ds(r0, rows_t), :],
                            acc.at[pl.ds(r0, rows_t), :])
            plsc.subcore_barrier()

            def body(i, _):
                m = c * nmb_core + i * NS + s
                base = m * MB
                pltpu.sync_copy(ei.at[which, pl.ds(base, MB)], idx_v)
                pltpu.sync_copy(ones_v, acc.at[idx_v], add=True)
                return 0

            lax.fori_loop(0, nit, body, 0)
            plsc.subcore_barrier()
            pltpu.sync_copy(acc.at[pl.ds(r0, rows_t), :],
                            out.at[c, which, pl.ds(r0, rows_t), :])

        run_pass(0)
        plsc.subcore_barrier()
        run_pass(1)

    return deg_kernel(edge_index, zpad, ones128)


def _agg_edge_loop(tab, ei, acc, sidx2, didx2, rows0, rows1,
                   gsem0, gsem1, isem0, isem1, mb0_of, nit):
    def start_idx(i, buf, isem):
        base = mb0_of(i) * MB
        pltpu.async_copy(ei.at[0, pl.ds(base, MB)], sidx2.at[buf], isem)
        pltpu.async_copy(ei.at[1, pl.ds(base, MB)], didx2.at[buf], isem)

    def wait_idx(i, buf, isem):
        base = mb0_of(i) * MB
        pltpu.make_async_copy(ei.at[0, pl.ds(base, MB)], sidx2.at[buf],
                              isem).wait()
        pltpu.make_async_copy(ei.at[1, pl.ds(base, MB)], didx2.at[buf],
                              isem).wait()

    def start_gather(buf_idx, rows, gsem):
        pltpu.async_copy(tab.at[sidx2.at[buf_idx]], rows, gsem)

    def wait_gather(buf_idx, rows, gsem):
        pltpu.make_async_copy(tab.at[sidx2.at[buf_idx]], rows, gsem).wait()

    start_idx(0, 0, isem0)
    start_idx(1, 1, isem1)
    wait_idx(0, 0, isem0)
    start_gather(0, rows0, gsem0)

    def body(g, _):
        i = 2 * g
        wait_idx(i + 1, 1, isem1)
        start_gather(1, rows1, gsem1)
        wait_gather(0, rows0, gsem0)
        pltpu.sync_copy(rows0, acc.at[didx2.at[0]], add=True)

        @pl.when(i + 2 < nit)
        def _():
            start_idx(i + 2, 0, isem0)
            wait_idx(i + 2, 0, isem0)
            start_gather(0, rows0, gsem0)

        wait_gather(1, rows1, gsem1)
        pltpu.sync_copy(rows1, acc.at[didx2.at[1]], add=True)

        @pl.when(i + 3 < nit)
        def _():
            start_idx(i + 3, 1, isem1)

        return 0

    lax.fori_loop(0, nit // 2, body, 0)


def _sc_agg128(table, edge_index, zpad, NP):
    E = edge_index.shape[1]
    nmb = E // MB
    nmb_core = nmb // NC
    rows_t = NP // NS
    nit = nmb_core // NS

    @functools.partial(
        pl.kernel,
        out_type=jax.ShapeDtypeStruct((NC, NP, 128), jnp.float32),
        mesh=_MESH,
        scratch_types=[
            pltpu.VMEM((2, MB), jnp.int32),
            pltpu.VMEM((2, MB), jnp.int32),
            pltpu.VMEM((MB, 128), jnp.float32),
            pltpu.VMEM((MB, 128), jnp.float32),
            pltpu.VMEM_SHARED((NP, 128), jnp.float32),
            pltpu.SemaphoreType.DMA,
            pltpu.SemaphoreType.DMA,
            pltpu.SemaphoreType.DMA,
            pltpu.SemaphoreType.DMA,
        ],
    )
    def agg_kernel(tab, ei, zp, out, sidx2, didx2, rows0, rows1, acc,
                   gsem0, gsem1, isem0, isem1):
        c = lax.axis_index("c")
        s = lax.axis_index("s")
        r0 = s * rows_t
        pltpu.sync_copy(zp.at[pl.ds(r0, rows_t), :],
                        acc.at[pl.ds(r0, rows_t), :])
        plsc.subcore_barrier()
        _agg_edge_loop(tab, ei, acc, sidx2, didx2, rows0, rows1,
                       gsem0, gsem1, isem0, isem1,
                       lambda i: c * nmb_core + i * NS + s, nit)
        plsc.subcore_barrier()
        pltpu.sync_copy(acc.at[pl.ds(r0, rows_t), :],
                        out.at[c, pl.ds(r0, rows_t), :])

    return agg_kernel(table, edge_index, zpad)


def _sc_agg512(t0, t1, t2, t3, edge_index, zpad, NP):
    E = edge_index.shape[1]
    nmb = E // MB
    rows_t = NP // NS
    ot = jax.ShapeDtypeStruct((NP, 128), jnp.float32)

    nit = nmb // NS

    @functools.partial(
        pl.kernel,
        out_type=(ot, ot, ot, ot),
        mesh=_MESH,
        scratch_types=[
            pltpu.VMEM((2, MB), jnp.int32),
            pltpu.VMEM((2, MB), jnp.int32),
            pltpu.VMEM((MB, 128), jnp.float32),
            pltpu.VMEM((MB, 128), jnp.float32),
            pltpu.VMEM_SHARED((NP, 128), jnp.float32),
            pltpu.SemaphoreType.DMA,
            pltpu.SemaphoreType.DMA,
            pltpu.SemaphoreType.DMA,
            pltpu.SemaphoreType.DMA,
        ],
    )
    def agg_kernel(a0, a1, a2, a3, ei, zp, o0, o1, o2, o3,
                   sidx2, didx2, rows0, rows1, acc,
                   gsem0, gsem1, isem0, isem1):
        c = lax.axis_index("c")
        s = lax.axis_index("s")
        r0 = s * rows_t

        def run_chunk(tab, out):
            pltpu.sync_copy(zp.at[pl.ds(r0, rows_t), :],
                            acc.at[pl.ds(r0, rows_t), :])
            plsc.subcore_barrier()
            _agg_edge_loop(tab, ei, acc, sidx2, didx2, rows0, rows1,
                           gsem0, gsem1, isem0, isem1,
                           lambda i: i * NS + s, nit)
            plsc.subcore_barrier()
            pltpu.sync_copy(acc.at[pl.ds(r0, rows_t), :],
                            out.at[pl.ds(r0, rows_t), :])

        @pl.when(c == 0)
        def _():
            run_chunk(a0, o0)
            plsc.subcore_barrier()
            run_chunk(a1, o1)

        @pl.when(c == 1)
        def _():
            run_chunk(a2, o2)
            plsc.subcore_barrier()
            run_chunk(a3, o3)

    return agg_kernel(t0, t1, t2, t3, edge_index, zpad)


def _tc_prep(x_pad, degp, maskcol, token, NP, MBK):
    grid = NP // MBK

    def body(x_ref, deg_ref, m_ref, tok_ref, oxn_ref, ni_ref, no_ref, mns_ref):
        dego = deg_ref[0, 0, :, 0:1] + deg_ref[1, 0, :, 0:1]
        degi = deg_ref[0, 1, :, 0:1] + deg_ref[1, 1, :, 0:1]
        no = jnp.where(dego > 0, lax.rsqrt(jnp.maximum(dego, 1e-30)), 0.0)
        ni = jnp.where(degi > 0, lax.rsqrt(jnp.maximum(degi, 1e-30)), 0.0)
        m = m_ref[...]
        ox = x_ref[...] * m + (1.0 - m) * tok_ref[...]
        oxn_ref[...] = ox * no
        ni_ref[...] = ni
        no_ref[...] = no
        mns_ref[...] = m * no

    return pl.pallas_call(
        body,
        grid=(grid,),
        in_specs=[
            pl.BlockSpec((MBK, 128), lambda i: (i, 0)),
            pl.BlockSpec((2, 2, MBK, 128), lambda i: (0, 0, i, 0)),
            pl.BlockSpec((MBK, 1), lambda i: (i, 0)),
            pl.BlockSpec((1, 128), lambda i: (0, 0)),
        ],
        out_specs=[
            pl.BlockSpec((MBK, 128), lambda i: (i, 0)),
            pl.BlockSpec((MBK, 1), lambda i: (i, 0)),
            pl.BlockSpec((MBK, 1), lambda i: (i, 0)),
            pl.BlockSpec((MBK, 1), lambda i: (i, 0)),
        ],
        out_shape=[
            jax.ShapeDtypeStruct((NP, 128), jnp.float32),
            jax.ShapeDtypeStruct((NP, 1), jnp.float32),
            jax.ShapeDtypeStruct((NP, 1), jnp.float32),
            jax.ShapeDtypeStruct((NP, 1), jnp.float32),
        ],
    )(x_pad, degp, maskcol, token)


def _tc_wed(W_e2d, W_dec):
    def body(a_ref, b_ref, o_ref):
        o_ref[...] = jnp.dot(a_ref[...], b_ref[...],
                             preferred_element_type=jnp.float32)

    return pl.pallas_call(
        body,
        out_shape=jax.ShapeDtypeStruct((512, 128), jnp.float32),
    )(W_e2d, W_dec)


def _tc_layer1(agg1, W1, b1, normin, normout, NP, MBK):
    grid = (NP // MBK, 4)

    def body(a_ref, w_ref, b_ref, ni_ref, no_ref, o_ref):
        a = a_ref[0] + a_ref[1]
        acc = jnp.dot(a, w_ref[...], preferred_element_type=jnp.float32)
        h = jnp.maximum(acc * ni_ref[...] + b_ref[...], 0.0)
        o_ref[0] = h * no_ref[...]

    return pl.pallas_call(
        body,
        grid=grid,
        in_specs=[
            pl.BlockSpec((2, MBK, 128), lambda i, c: (0, i, 0)),
            pl.BlockSpec((128, 128), lambda i, c: (0, c)),
            pl.BlockSpec((1, 128), lambda i, c: (0, c)),
            pl.BlockSpec((MBK, 1), lambda i, c: (i, 0)),
            pl.BlockSpec((MBK, 1), lambda i, c: (i, 0)),
        ],
        out_specs=pl.BlockSpec((1, MBK, 128), lambda i, c: (c, i, 0)),
        out_shape=jax.ShapeDtypeStruct((4, NP, 128), jnp.float32),
    )(agg1, W1, b1, normin, normout)


def _tc_layer2(agg2, W2, b2, normin, mns, W_ed, NP, MBK):
    grid = (NP // MBK,)

    def body(a_ref, w_ref, b_ref, ni_ref, mns_ref, wed_ref, enc_ref, d_ref):
        acc = jnp.dot(a_ref[0], w_ref[pl.ds(0, 128), :],
                      preferred_element_type=jnp.float32)
        for cc in range(1, 4):
            acc += jnp.dot(a_ref[cc], w_ref[pl.ds(cc * 128, 128), :],
                           preferred_element_type=jnp.float32)
        enc = jnp.maximum(acc * ni_ref[...] + b_ref[...], 0.0)
        enc_ref[...] = enc
        d_ref[...] = jnp.dot(enc * mns_ref[...], wed_ref[...],
                             preferred_element_type=jnp.float32)

    return pl.pallas_call(
        body,
        grid=grid,
        in_specs=[
            pl.BlockSpec((4, MBK, 128), lambda i: (0, i, 0)),
            pl.BlockSpec((512, 512), lambda i: (0, 0)),
            pl.BlockSpec((1, 512), lambda i: (0, 0)),
            pl.BlockSpec((MBK, 1), lambda i: (i, 0)),
            pl.BlockSpec((MBK, 1), lambda i: (i, 0)),
            pl.BlockSpec((512, 128), lambda i: (0, 0)),
        ],
        out_specs=[
            pl.BlockSpec((MBK, 512), lambda i: (i, 0)),
            pl.BlockSpec((MBK, 128), lambda i: (i, 0)),
        ],
        out_shape=[
            jax.ShapeDtypeStruct((NP, 512), jnp.float32),
            jax.ShapeDtypeStruct((NP, 128), jnp.float32),
        ],
    )(agg2, W2, b2, normin, mns, W_ed)


def _tc_final(agg3, b_dec, normin, maskcol, x_pad, NP, MBK):
    grid = (NP // MBK,)

    def body(a_ref, b_ref, ni_ref, m_ref, x_ref, rec_ref, loss_ref):
        i = pl.program_id(0)
        r = (a_ref[0] + a_ref[1]) * ni_ref[...] + b_ref[...]
        rec_ref[...] = r
        w = 1.0 - m_ref[...]
        x = x_ref[...]
        rnorm = jnp.sqrt(jnp.sum(r * r, axis=-1, keepdims=True))
        xnorm = jnp.sqrt(jnp.sum(x * x, axis=-1, keepdims=True))
        rn = r / jnp.maximum(rnorm, 1e-12)
        xn = x / jnp.maximum(xnorm, 1e-12)
        cos = jnp.sum(rn * xn, axis=-1, keepdims=True)
        contrib = jnp.sum(w * (1.0 - cos) ** 2, keepdims=True).reshape(1, 1)

        @pl.when(i == 0)
        def _():
            loss_ref[...] = contrib

        @pl.when(i > 0)
        def _():
            loss_ref[...] += contrib

    return pl.pallas_call(
        body,
        grid=grid,
        in_specs=[
            pl.BlockSpec((2, MBK, 128), lambda i: (0, i, 0)),
            pl.BlockSpec((1, 128), lambda i: (0, 0)),
            pl.BlockSpec((MBK, 1), lambda i: (i, 0)),
            pl.BlockSpec((MBK, 1), lambda i: (i, 0)),
            pl.BlockSpec((MBK, 128), lambda i: (i, 0)),
        ],
        out_specs=[
            pl.BlockSpec((MBK, 128), lambda i: (i, 0)),
            pl.BlockSpec((1, 1), lambda i: (0, 0)),
        ],
        out_shape=[
            jax.ShapeDtypeStruct((NP, 128), jnp.float32),
            jax.ShapeDtypeStruct((1, 1), jnp.float32),
        ],
    )(agg3, b_dec, normin, maskcol, x_pad)


def kernel(x, edge_index, mask_nodes, enc_mask_token,
           W_enc1, b_enc1, W_enc2, b_enc2, W_e2d, W_dec, b_dec):
    N = x.shape[0]
    num_mask = mask_nodes.shape[0]
    NP = ((N + NS * 40 - 1) // (NS * 40)) * (NS * 40)
    MBK = NP // 8

    x_pad = jnp.pad(x, ((0, NP - N), (0, 0)))
    maskcol = jnp.ones((NP, 1), jnp.float32).at[mask_nodes].set(0.0)
    zpad = jnp.zeros((NP, 128), jnp.float32)
    ones128 = jnp.ones((MB, 128), jnp.float32)

    E = edge_index.shape[1]
    EDIV = MB * NS * NC * 2
    EP = ((E + EDIV - 1) // EDIV) * EDIV
    pad_nodes = N + jnp.arange(EP - E, dtype=jnp.int32) % (NP - N)
    edge_index = jnp.concatenate(
        [edge_index, jnp.stack([pad_nodes, pad_nodes])], axis=1)

    degp = _sc_degrees(edge_index, zpad, ones128, NP)
    oxn, normin, normout, mns = _tc_prep(
        x_pad, degp, maskcol, enc_mask_token, NP, MBK)
    W_ed = _tc_wed(W_e2d, W_dec)

    agg1 = _sc_agg128(oxn, edge_index, zpad, NP)
    h1n = _tc_layer1(agg1, W_enc1, b_enc1.reshape(1, -1), normin, normout,
                     NP, MBK)
    agg2c = _sc_agg512(h1n[0], h1n[1], h1n[2], h1n[3], edge_index, zpad, NP)
    agg2 = jnp.stack(agg2c)
    enc_pad, d = _tc_layer2(agg2, W_enc2, b_enc2.reshape(1, -1), normin, mns,
                            W_ed, NP, MBK)
    agg3 = _sc_agg128(d, edge_index, zpad, NP)
    recon_pad, loss_acc = _tc_final(agg3, b_dec.reshape(1, -1), normin,
                                    maskcol, x_pad, NP, MBK)

    enc_rep = enc_pad[:N]
    recon = recon_pad[:N]
    loss = (loss_acc[0, 0] / num_mask).astype(jnp.float32)
    return (enc_rep, recon, loss)


# speedup vs baseline: 2.3123x; 1.0646x over previous
import functools

import jax
import jax.numpy as jnp
from jax import lax
from jax.experimental import pallas as pl
from jax.experimental.pallas import tpu as pltpu
from jax.experimental.pallas import tpu_sc as plsc

NC = 2
NS = 16
MB = 128

_MESH = plsc.VectorSubcoreMesh(
    core_axis_name="c", subcore_axis_name="s", num_cores=NC, num_subcores=NS)


def _sc_degrees(edge_index, zpad, ones128, NP):
    E = edge_index.shape[1]
    nmb = E // MB
    nmb_core = nmb // NC
    rows_t = NP // NS
    nit = nmb_core // NS

    @functools.partial(
        pl.kernel,
        out_type=jax.ShapeDtypeStruct((NC, 2, NP, 128), jnp.float32),
        mesh=_MESH,
        scratch_types=[
            pltpu.VMEM((2, MB), jnp.int32),
            pltpu.VMEM((MB, 128), jnp.float32),
            pltpu.VMEM_SHARED((NP, 128), jnp.float32),
            pltpu.SemaphoreType.DMA,
            pltpu.SemaphoreType.DMA,
        ],
    )
    def deg_kernel(ei, zp, ones_hbm, out, idx2, ones_v, acc, isem0, isem1):
        c = lax.axis_index("c")
        s = lax.axis_index("s")
        r0 = s * rows_t
        pltpu.sync_copy(ones_hbm, ones_v)

        def run_pass(which):
            pltpu.sync_copy(zp.at[pl.ds(r0, rows_t), :],
                            acc.at[pl.ds(r0, rows_t), :])
            plsc.subcore_barrier()

            def start_idx(i, buf, isem):
                base = (c * nmb_core + i * NS + s) * MB
                pltpu.async_copy(ei.at[which, pl.ds(base, MB)],
                                 idx2.at[buf], isem)

            def wait_idx(i, buf, isem):
                base = (c * nmb_core + i * NS + s) * MB
                pltpu.make_async_copy(ei.at[which, pl.ds(base, MB)],
                                      idx2.at[buf], isem).wait()

            start_idx(0, 0, isem0)
            start_idx(1, 1, isem1)

            def body(g, _):
                i = 2 * g
                wait_idx(i, 0, isem0)
                pltpu.sync_copy(ones_v, acc.at[idx2.at[0]], add=True)

                @pl.when(i + 2 < nit)
                def _():
                    start_idx(i + 2, 0, isem0)

                wait_idx(i + 1, 1, isem1)
                pltpu.sync_copy(ones_v, acc.at[idx2.at[1]], add=True)

                @pl.when(i + 3 < nit)
                def _():
                    start_idx(i + 3, 1, isem1)

                return 0

            lax.fori_loop(0, nit // 2, body, 0)
            plsc.subcore_barrier()
            pltpu.sync_copy(acc.at[pl.ds(r0, rows_t), :],
                            out.at[c, which, pl.ds(r0, rows_t), :])

        run_pass(0)
        plsc.subcore_barrier()
        run_pass(1)

    return deg_kernel(edge_index, zpad, ones128)


def _agg_edge_loop(tab, ei, acc, sidx2, didx2, rows0, rows1,
                   gsem0, gsem1, isem0, isem1, mb0_of, nit):
    def start_idx(i, buf, isem):
        base = mb0_of(i) * MB
        pltpu.async_copy(ei.at[0, pl.ds(base, MB)], sidx2.at[buf], isem)
        pltpu.async_copy(ei.at[1, pl.ds(base, MB)], didx2.at[buf], isem)

    def wait_idx(i, buf, isem):
        base = mb0_of(i) * MB
        pltpu.make_async_copy(ei.at[0, pl.ds(base, MB)], sidx2.at[buf],
                              isem).wait()
        pltpu.make_async_copy(ei.at[1, pl.ds(base, MB)], didx2.at[buf],
                              isem).wait()

    def start_gather(buf_idx, rows, gsem):
        pltpu.async_copy(tab.at[sidx2.at[buf_idx]], rows, gsem)

    def wait_gather(buf_idx, rows, gsem):
        pltpu.make_async_copy(tab.at[sidx2.at[buf_idx]], rows, gsem).wait()

    start_idx(0, 0, isem0)
    start_idx(1, 1, isem1)
    wait_idx(0, 0, isem0)
    start_gather(0, rows0, gsem0)

    def body(g, _):
        i = 2 * g
        wait_idx(i + 1, 1, isem1)
        start_gather(1, rows1, gsem1)
        wait_gather(0, rows0, gsem0)
        pltpu.sync_copy(rows0, acc.at[didx2.at[0]], add=True)

        @pl.when(i + 2 < nit)
        def _():
            start_idx(i + 2, 0, isem0)
            wait_idx(i + 2, 0, isem0)
            start_gather(0, rows0, gsem0)

        wait_gather(1, rows1, gsem1)
        pltpu.sync_copy(rows1, acc.at[didx2.at[1]], add=True)

        @pl.when(i + 3 < nit)
        def _():
            start_idx(i + 3, 1, isem1)

        return 0

    lax.fori_loop(0, nit // 2, body, 0)


def _sc_agg128(table, edge_index, zpad, NP):
    E = edge_index.shape[1]
    nmb = E // MB
    nmb_core = nmb // NC
    rows_t = NP // NS
    nit = nmb_core // NS

    @functools.partial(
        pl.kernel,
        out_type=jax.ShapeDtypeStruct((NC, NP, 128), jnp.float32),
        mesh=_MESH,
        scratch_types=[
            pltpu.VMEM((2, MB), jnp.int32),
            pltpu.VMEM((2, MB), jnp.int32),
            pltpu.VMEM((MB, 128), jnp.float32),
            pltpu.VMEM((MB, 128), jnp.float32),
            pltpu.VMEM_SHARED((NP, 128), jnp.float32),
            pltpu.SemaphoreType.DMA,
            pltpu.SemaphoreType.DMA,
            pltpu.SemaphoreType.DMA,
            pltpu.SemaphoreType.DMA,
        ],
    )
    def agg_kernel(tab, ei, zp, out, sidx2, didx2, rows0, rows1, acc,
                   gsem0, gsem1, isem0, isem1):
        c = lax.axis_index("c")
        s = lax.axis_index("s")
        r0 = s * rows_t
        pltpu.sync_copy(zp.at[pl.ds(r0, rows_t), :],
                        acc.at[pl.ds(r0, rows_t), :])
        plsc.subcore_barrier()
        _agg_edge_loop(tab, ei, acc, sidx2, didx2, rows0, rows1,
                       gsem0, gsem1, isem0, isem1,
                       lambda i: c * nmb_core + i * NS + s, nit)
        plsc.subcore_barrier()
        pltpu.sync_copy(acc.at[pl.ds(r0, rows_t), :],
                        out.at[c, pl.ds(r0, rows_t), :])

    return agg_kernel(table, edge_index, zpad)


def _sc_agg512(t0, t1, t2, t3, edge_index, zpad, NP):
    E = edge_index.shape[1]
    nmb = E // MB
    rows_t = NP // NS
    ot = jax.ShapeDtypeStruct((NP, 128), jnp.float32)

    nit = nmb // NS

    @functools.partial(
        pl.kernel,
        out_type=(ot, ot, ot, ot),
        mesh=_MESH,
        scratch_types=[
            pltpu.VMEM((2, MB), jnp.int32),
            pltpu.VMEM((2, MB), jnp.int32),
            pltpu.VMEM((MB, 128), jnp.float32),
            pltpu.VMEM((MB, 128), jnp.float32),
            pltpu.VMEM_SHARED((NP, 128), jnp.float32),
            pltpu.SemaphoreType.DMA,
            pltpu.SemaphoreType.DMA,
            pltpu.SemaphoreType.DMA,
            pltpu.SemaphoreType.DMA,
        ],
    )
    def agg_kernel(a0, a1, a2, a3, ei, zp, o0, o1, o2, o3,
                   sidx2, didx2, rows0, rows1, acc,
                   gsem0, gsem1, isem0, isem1):
        c = lax.axis_index("c")
        s = lax.axis_index("s")
        r0 = s * rows_t

        def run_chunk(tab, out):
            pltpu.sync_copy(zp.at[pl.ds(r0, rows_t), :],
                            acc.at[pl.ds(r0, rows_t), :])
            plsc.subcore_barrier()
            _agg_edge_loop(tab, ei, acc, sidx2, didx2, rows0, rows1,
                           gsem0, gsem1, isem0, isem1,
                           lambda i: i * NS + s, nit)
            plsc.subcore_barrier()
            pltpu.sync_copy(acc.at[pl.ds(r0, rows_t), :],
                            out.at[pl.ds(r0, rows_t), :])

        @pl.when(c == 0)
        def _():
            run_chunk(a0, o0)
            plsc.subcore_barrier()
            run_chunk(a1, o1)

        @pl.when(c == 1)
        def _():
            run_chunk(a2, o2)
            plsc.subcore_barrier()
            run_chunk(a3, o3)

    return agg_kernel(t0, t1, t2, t3, edge_index, zpad)


def _tc_prep(x_pad, degp, maskcol, token, NP, MBK):
    grid = NP // MBK

    def body(x_ref, deg_ref, m_ref, tok_ref, oxn_ref, ni_ref, no_ref, mns_ref):
        dego = deg_ref[0, 0, :, 0:1] + deg_ref[1, 0, :, 0:1]
        degi = deg_ref[0, 1, :, 0:1] + deg_ref[1, 1, :, 0:1]
        no = jnp.where(dego > 0, lax.rsqrt(jnp.maximum(dego, 1e-30)), 0.0)
        ni = jnp.where(degi > 0, lax.rsqrt(jnp.maximum(degi, 1e-30)), 0.0)
        m = m_ref[...]
        ox = x_ref[...] * m + (1.0 - m) * tok_ref[...]
        oxn_ref[...] = ox * no
        ni_ref[...] = ni
        no_ref[...] = no
        mns_ref[...] = m * no

    return pl.pallas_call(
        body,
        grid=(grid,),
        in_specs=[
            pl.BlockSpec((MBK, 128), lambda i: (i, 0)),
            pl.BlockSpec((2, 2, MBK, 128), lambda i: (0, 0, i, 0)),
            pl.BlockSpec((MBK, 1), lambda i: (i, 0)),
            pl.BlockSpec((1, 128), lambda i: (0, 0)),
        ],
        out_specs=[
            pl.BlockSpec((MBK, 128), lambda i: (i, 0)),
            pl.BlockSpec((MBK, 1), lambda i: (i, 0)),
            pl.BlockSpec((MBK, 1), lambda i: (i, 0)),
            pl.BlockSpec((MBK, 1), lambda i: (i, 0)),
        ],
        out_shape=[
            jax.ShapeDtypeStruct((NP, 128), jnp.float32),
            jax.ShapeDtypeStruct((NP, 1), jnp.float32),
            jax.ShapeDtypeStruct((NP, 1), jnp.float32),
            jax.ShapeDtypeStruct((NP, 1), jnp.float32),
        ],
    )(x_pad, degp, maskcol, token)


def _tc_wed(W_e2d, W_dec):
    def body(a_ref, b_ref, o_ref):
        o_ref[...] = jnp.dot(a_ref[...], b_ref[...],
                             preferred_element_type=jnp.float32)

    return pl.pallas_call(
        body,
        out_shape=jax.ShapeDtypeStruct((512, 128), jnp.float32),
    )(W_e2d, W_dec)


def _tc_layer1(agg1, W1, b1, normin, normout, NP, MBK):
    grid = (NP // MBK, 4)

    def body(a_ref, w_ref, b_ref, ni_ref, no_ref, o_ref):
        a = a_ref[0] + a_ref[1]
        acc = jnp.dot(a, w_ref[...], preferred_element_type=jnp.float32)
        h = jnp.maximum(acc * ni_ref[...] + b_ref[...], 0.0)
        o_ref[0] = h * no_ref[...]

    return pl.pallas_call(
        body,
        grid=grid,
        in_specs=[
            pl.BlockSpec((2, MBK, 128), lambda i, c: (0, i, 0)),
            pl.BlockSpec((128, 128), lambda i, c: (0, c)),
            pl.BlockSpec((1, 128), lambda i, c: (0, c)),
            pl.BlockSpec((MBK, 1), lambda i, c: (i, 0)),
            pl.BlockSpec((MBK, 1), lambda i, c: (i, 0)),
        ],
        out_specs=pl.BlockSpec((1, MBK, 128), lambda i, c: (c, i, 0)),
        out_shape=jax.ShapeDtypeStruct((4, NP, 128), jnp.float32),
    )(agg1, W1, b1, normin, normout)


def _tc_layer2(agg2, W2, b2, normin, mns, W_ed, NP, MBK):
    grid = (NP // MBK,)

    def body(a_ref, w_ref, b_ref, ni_ref, mns_ref, wed_ref, enc_ref, d_ref):
        acc = jnp.dot(a_ref[0], w_ref[pl.ds(0, 128), :],
                      preferred_element_type=jnp.float32)
        for cc in range(1, 4):
            acc += jnp.dot(a_ref[cc], w_ref[pl.ds(cc * 128, 128), :],
                           preferred_element_type=jnp.float32)
        enc = jnp.maximum(acc * ni_ref[...] + b_ref[...], 0.0)
        enc_ref[...] = enc
        d_ref[...] = jnp.dot(enc * mns_ref[...], wed_ref[...],
                             preferred_element_type=jnp.float32)

    return pl.pallas_call(
        body,
        grid=grid,
        in_specs=[
            pl.BlockSpec((4, MBK, 128), lambda i: (0, i, 0)),
            pl.BlockSpec((512, 512), lambda i: (0, 0)),
            pl.BlockSpec((1, 512), lambda i: (0, 0)),
            pl.BlockSpec((MBK, 1), lambda i: (i, 0)),
            pl.BlockSpec((MBK, 1), lambda i: (i, 0)),
            pl.BlockSpec((512, 128), lambda i: (0, 0)),
        ],
        out_specs=[
            pl.BlockSpec((MBK, 512), lambda i: (i, 0)),
            pl.BlockSpec((MBK, 128), lambda i: (i, 0)),
        ],
        out_shape=[
            jax.ShapeDtypeStruct((NP, 512), jnp.float32),
            jax.ShapeDtypeStruct((NP, 128), jnp.float32),
        ],
    )(agg2, W2, b2, normin, mns, W_ed)


def _tc_final(agg3, b_dec, normin, maskcol, x_pad, NP, MBK):
    grid = (NP // MBK,)

    def body(a_ref, b_ref, ni_ref, m_ref, x_ref, rec_ref, loss_ref):
        i = pl.program_id(0)
        r = (a_ref[0] + a_ref[1]) * ni_ref[...] + b_ref[...]
        rec_ref[...] = r
        w = 1.0 - m_ref[...]
        x = x_ref[...]
        rnorm = jnp.sqrt(jnp.sum(r * r, axis=-1, keepdims=True))
        xnorm = jnp.sqrt(jnp.sum(x * x, axis=-1, keepdims=True))
        rn = r / jnp.maximum(rnorm, 1e-12)
        xn = x / jnp.maximum(xnorm, 1e-12)
        cos = jnp.sum(rn * xn, axis=-1, keepdims=True)
        contrib = jnp.sum(w * (1.0 - cos) ** 2, keepdims=True).reshape(1, 1)

        @pl.when(i == 0)
        def _():
            loss_ref[...] = contrib

        @pl.when(i > 0)
        def _():
            loss_ref[...] += contrib

    return pl.pallas_call(
        body,
        grid=grid,
        in_specs=[
            pl.BlockSpec((2, MBK, 128), lambda i: (0, i, 0)),
            pl.BlockSpec((1, 128), lambda i: (0, 0)),
            pl.BlockSpec((MBK, 1), lambda i: (i, 0)),
            pl.BlockSpec((MBK, 1), lambda i: (i, 0)),
            pl.BlockSpec((MBK, 128), lambda i: (i, 0)),
        ],
        out_specs=[
            pl.BlockSpec((MBK, 128), lambda i: (i, 0)),
            pl.BlockSpec((1, 1), lambda i: (0, 0)),
        ],
        out_shape=[
            jax.ShapeDtypeStruct((NP, 128), jnp.float32),
            jax.ShapeDtypeStruct((1, 1), jnp.float32),
        ],
    )(agg3, b_dec, normin, maskcol, x_pad)


def kernel(x, edge_index, mask_nodes, enc_mask_token,
           W_enc1, b_enc1, W_enc2, b_enc2, W_e2d, W_dec, b_dec):
    N = x.shape[0]
    num_mask = mask_nodes.shape[0]
    NP = ((N + NS * 40 - 1) // (NS * 40)) * (NS * 40)
    MBK = NP // 8

    x_pad = jnp.pad(x, ((0, NP - N), (0, 0)))
    maskcol = jnp.ones((NP, 1), jnp.float32).at[mask_nodes].set(0.0)
    zpad = jnp.zeros((NP, 128), jnp.float32)
    ones128 = jnp.ones((MB, 128), jnp.float32)

    E = edge_index.shape[1]
    EDIV = MB * NS * NC * 2
    EP = ((E + EDIV - 1) // EDIV) * EDIV
    pad_nodes = N + jnp.arange(EP - E, dtype=jnp.int32) % (NP - N)
    edge_index = jnp.concatenate(
        [edge_index, jnp.stack([pad_nodes, pad_nodes])], axis=1)

    degp = _sc_degrees(edge_index, zpad, ones128, NP)
    oxn, normin, normout, mns = _tc_prep(
        x_pad, degp, maskcol, enc_mask_token, NP, MBK)
    W_ed = _tc_wed(W_e2d, W_dec)

    agg1 = _sc_agg128(oxn, edge_index, zpad, NP)
    h1n = _tc_layer1(agg1, W_enc1, b_enc1.reshape(1, -1), normin, normout,
                     NP, MBK)
    agg2c = _sc_agg512(h1n[0], h1n[1], h1n[2], h1n[3], edge_index, zpad, NP)
    agg2 = jnp.stack(agg2c)
    enc_pad, d = _tc_layer2(agg2, W_enc2, b_enc2.reshape(1, -1), normin, mns,
                            W_ed, NP, MBK)
    agg3 = _sc_agg128(d, edge_index, zpad, NP)
    recon_pad, loss_acc = _tc_final(agg3, b_dec.reshape(1, -1), normin,
                                    maskcol, x_pad, NP, MBK)

    enc_rep = enc_pad[:N]
    recon = recon_pad[:N]
    loss = (loss_acc[0, 0] / num_mask).astype(jnp.float32)
    return (enc_rep, recon, loss)


# speedup vs baseline: 2.3857x; 1.0317x over previous
import functools

import jax
import jax.numpy as jnp
from jax import lax
from jax.experimental import pallas as pl
from jax.experimental.pallas import tpu as pltpu
from jax.experimental.pallas import tpu_sc as plsc

NC = 2
NS = 16
MB = 128

_MESH = plsc.VectorSubcoreMesh(
    core_axis_name="c", subcore_axis_name="s", num_cores=NC, num_subcores=NS)


def _sc_degrees(edge_index, zpad, ones128, NP):
    E = edge_index.shape[1]
    nmb = E // MB
    nmb_core = nmb // NC
    rows_t = NP // NS
    nit = nmb_core // NS

    nit_all = nmb // NS

    @functools.partial(
        pl.kernel,
        out_type=jax.ShapeDtypeStruct((2, NP, 128), jnp.float32),
        mesh=_MESH,
        scratch_types=[
            pltpu.VMEM((2, MB), jnp.int32),
            pltpu.VMEM((MB, 128), jnp.float32),
            pltpu.VMEM_SHARED((NP, 128), jnp.float32),
            pltpu.SemaphoreType.DMA,
            pltpu.SemaphoreType.DMA,
        ],
    )
    def deg_kernel(ei, zp, ones_hbm, out, idx2, ones_v, acc, isem0, isem1):
        c = lax.axis_index("c")
        s = lax.axis_index("s")
        r0 = s * rows_t
        pltpu.sync_copy(ones_hbm, ones_v)
        pltpu.sync_copy(zp.at[pl.ds(r0, rows_t), :],
                        acc.at[pl.ds(r0, rows_t), :])
        plsc.subcore_barrier()

        def run_pass(which):
            def start_idx(i, buf, isem):
                base = (i * NS + s) * MB
                pltpu.async_copy(ei.at[which, pl.ds(base, MB)],
                                 idx2.at[buf], isem)

            def wait_idx(i, buf, isem):
                base = (i * NS + s) * MB
                pltpu.make_async_copy(ei.at[which, pl.ds(base, MB)],
                                      idx2.at[buf], isem).wait()

            start_idx(0, 0, isem0)
            start_idx(1, 1, isem1)

            def body(g, _):
                i = 2 * g
                wait_idx(i, 0, isem0)
                pltpu.sync_copy(ones_v, acc.at[idx2.at[0]], add=True)

                @pl.when(i + 2 < nit_all)
                def _():
                    start_idx(i + 2, 0, isem0)

                wait_idx(i + 1, 1, isem1)
                pltpu.sync_copy(ones_v, acc.at[idx2.at[1]], add=True)

                @pl.when(i + 3 < nit_all)
                def _():
                    start_idx(i + 3, 1, isem1)

                return 0

            lax.fori_loop(0, nit_all // 2, body, 0)
            plsc.subcore_barrier()
            pltpu.sync_copy(acc.at[pl.ds(r0, rows_t), :],
                            out.at[which, pl.ds(r0, rows_t), :])

        @pl.when(c == 0)
        def _():
            run_pass(0)

        @pl.when(c == 1)
        def _():
            run_pass(1)

    return deg_kernel(edge_index, zpad, ones128)


def _agg_edge_loop(tab, ei, acc, sidx2, didx2, rows0, rows1,
                   gsem0, gsem1, isem0, isem1, mb0_of, nit):
    def start_idx(i, buf, isem):
        base = mb0_of(i) * MB
        pltpu.async_copy(ei.at[0, pl.ds(base, MB)], sidx2.at[buf], isem)
        pltpu.async_copy(ei.at[1, pl.ds(base, MB)], didx2.at[buf], isem)

    def wait_idx(i, buf, isem):
        base = mb0_of(i) * MB
        pltpu.make_async_copy(ei.at[0, pl.ds(base, MB)], sidx2.at[buf],
                              isem).wait()
        pltpu.make_async_copy(ei.at[1, pl.ds(base, MB)], didx2.at[buf],
                              isem).wait()

    def start_gather(buf_idx, rows, gsem):
        pltpu.async_copy(tab.at[sidx2.at[buf_idx]], rows, gsem)

    def wait_gather(buf_idx, rows, gsem):
        pltpu.make_async_copy(tab.at[sidx2.at[buf_idx]], rows, gsem).wait()

    start_idx(0, 0, isem0)
    start_idx(1, 1, isem1)
    wait_idx(0, 0, isem0)
    start_gather(0, rows0, gsem0)

    def body(g, _):
        i = 2 * g
        wait_idx(i + 1, 1, isem1)
        start_gather(1, rows1, gsem1)
        wait_gather(0, rows0, gsem0)
        pltpu.sync_copy(rows0, acc.at[didx2.at[0]], add=True)

        @pl.when(i + 2 < nit)
        def _():
            start_idx(i + 2, 0, isem0)
            wait_idx(i + 2, 0, isem0)
            start_gather(0, rows0, gsem0)

        wait_gather(1, rows1, gsem1)
        pltpu.sync_copy(rows1, acc.at[didx2.at[1]], add=True)

        @pl.when(i + 3 < nit)
        def _():
            start_idx(i + 3, 1, isem1)

        return 0

    lax.fori_loop(0, nit // 2, body, 0)


def _sc_agg128(table, edge_index, zpad, NP):
    E = edge_index.shape[1]
    nmb = E // MB
    nmb_core = nmb // NC
    rows_t = NP // NS
    nit = nmb_core // NS

    @functools.partial(
        pl.kernel,
        out_type=jax.ShapeDtypeStruct((NC, NP, 128), jnp.float32),
        mesh=_MESH,
        scratch_types=[
            pltpu.VMEM((2, MB), jnp.int32),
            pltpu.VMEM((2, MB), jnp.int32),
            pltpu.VMEM((MB, 128), jnp.float32),
            pltpu.VMEM((MB, 128), jnp.float32),
            pltpu.VMEM_SHARED((NP, 128), jnp.float32),
            pltpu.SemaphoreType.DMA,
            pltpu.SemaphoreType.DMA,
            pltpu.SemaphoreType.DMA,
            pltpu.SemaphoreType.DMA,
        ],
    )
    def agg_kernel(tab, ei, zp, out, sidx2, didx2, rows0, rows1, acc,
                   gsem0, gsem1, isem0, isem1):
        c = lax.axis_index("c")
        s = lax.axis_index("s")
        r0 = s * rows_t
        pltpu.sync_copy(zp.at[pl.ds(r0, rows_t), :],
                        acc.at[pl.ds(r0, rows_t), :])
        plsc.subcore_barrier()
        _agg_edge_loop(tab, ei, acc, sidx2, didx2, rows0, rows1,
                       gsem0, gsem1, isem0, isem1,
                       lambda i: c * nmb_core + i * NS + s, nit)
        plsc.subcore_barrier()
        pltpu.sync_copy(acc.at[pl.ds(r0, rows_t), :],
                        out.at[c, pl.ds(r0, rows_t), :])

    return agg_kernel(table, edge_index, zpad)


def _sc_agg512(t0, t1, t2, t3, edge_index, zpad, NP):
    E = edge_index.shape[1]
    nmb = E // MB
    rows_t = NP // NS
    ot = jax.ShapeDtypeStruct((NP, 128), jnp.float32)

    nit = nmb // NS

    @functools.partial(
        pl.kernel,
        out_type=(ot, ot, ot, ot),
        mesh=_MESH,
        scratch_types=[
            pltpu.VMEM((2, MB), jnp.int32),
            pltpu.VMEM((2, MB), jnp.int32),
            pltpu.VMEM((MB, 128), jnp.float32),
            pltpu.VMEM((MB, 128), jnp.float32),
            pltpu.VMEM_SHARED((NP, 128), jnp.float32),
            pltpu.SemaphoreType.DMA,
            pltpu.SemaphoreType.DMA,
            pltpu.SemaphoreType.DMA,
            pltpu.SemaphoreType.DMA,
        ],
    )
    def agg_kernel(a0, a1, a2, a3, ei, zp, o0, o1, o2, o3,
                   sidx2, didx2, rows0, rows1, acc,
                   gsem0, gsem1, isem0, isem1):
        c = lax.axis_index("c")
        s = lax.axis_index("s")
        r0 = s * rows_t

        def run_chunk(tab, out):
            pltpu.sync_copy(zp.at[pl.ds(r0, rows_t), :],
                            acc.at[pl.ds(r0, rows_t), :])
            plsc.subcore_barrier()
            _agg_edge_loop(tab, ei, acc, sidx2, didx2, rows0, rows1,
                           gsem0, gsem1, isem0, isem1,
                           lambda i: i * NS + s, nit)
            plsc.subcore_barrier()
            pltpu.sync_copy(acc.at[pl.ds(r0, rows_t), :],
                            out.at[pl.ds(r0, rows_t), :])

        @pl.when(c == 0)
        def _():
            run_chunk(a0, o0)
            plsc.subcore_barrier()
            run_chunk(a1, o1)

        @pl.when(c == 1)
        def _():
            run_chunk(a2, o2)
            plsc.subcore_barrier()
            run_chunk(a3, o3)

    return agg_kernel(t0, t1, t2, t3, edge_index, zpad)


def _tc_prep(x_pad, degp, maskcol, token, NP, MBK):
    grid = NP // MBK

    def body(x_ref, deg_ref, m_ref, tok_ref, oxn_ref, ni_ref, no_ref, mns_ref):
        dego = deg_ref[0, :, 0:1]
        degi = deg_ref[1, :, 0:1]
        no = jnp.where(dego > 0, lax.rsqrt(jnp.maximum(dego, 1e-30)), 0.0)
        ni = jnp.where(degi > 0, lax.rsqrt(jnp.maximum(degi, 1e-30)), 0.0)
        m = m_ref[...]
        ox = x_ref[...] * m + (1.0 - m) * tok_ref[...]
        oxn_ref[...] = ox * no
        ni_ref[...] = ni
        no_ref[...] = no
        mns_ref[...] = m * no

    return pl.pallas_call(
        body,
        grid=(grid,),
        in_specs=[
            pl.BlockSpec((MBK, 128), lambda i: (i, 0)),
            pl.BlockSpec((2, MBK, 128), lambda i: (0, i, 0)),
            pl.BlockSpec((MBK, 1), lambda i: (i, 0)),
            pl.BlockSpec((1, 128), lambda i: (0, 0)),
        ],
        out_specs=[
            pl.BlockSpec((MBK, 128), lambda i: (i, 0)),
            pl.BlockSpec((MBK, 1), lambda i: (i, 0)),
            pl.BlockSpec((MBK, 1), lambda i: (i, 0)),
            pl.BlockSpec((MBK, 1), lambda i: (i, 0)),
        ],
        out_shape=[
            jax.ShapeDtypeStruct((NP, 128), jnp.float32),
            jax.ShapeDtypeStruct((NP, 1), jnp.float32),
            jax.ShapeDtypeStruct((NP, 1), jnp.float32),
            jax.ShapeDtypeStruct((NP, 1), jnp.float32),
        ],
    )(x_pad, degp, maskcol, token)


def _tc_layer1(agg1, W1, b1, normin, normout, NP, MBK):
    grid = (NP // MBK, 4)

    def body(a_ref, w_ref, b_ref, ni_ref, no_ref, o_ref):
        a = a_ref[0] + a_ref[1]
        acc = jnp.dot(a, w_ref[...], preferred_element_type=jnp.float32)
        h = jnp.maximum(acc * ni_ref[...] + b_ref[...], 0.0)
        o_ref[0] = h * no_ref[...]

    return pl.pallas_call(
        body,
        grid=grid,
        in_specs=[
            pl.BlockSpec((2, MBK, 128), lambda i, c: (0, i, 0)),
            pl.BlockSpec((128, 128), lambda i, c: (0, c)),
            pl.BlockSpec((1, 128), lambda i, c: (0, c)),
            pl.BlockSpec((MBK, 1), lambda i, c: (i, 0)),
            pl.BlockSpec((MBK, 1), lambda i, c: (i, 0)),
        ],
        out_specs=pl.BlockSpec((1, MBK, 128), lambda i, c: (c, i, 0)),
        out_shape=jax.ShapeDtypeStruct((4, NP, 128), jnp.float32),
    )(agg1, W1, b1, normin, normout)


def _tc_layer2(aggc, W2, b2, normin, mns, W_e2d, W_dec, NP, MBK):
    grid = (NP // MBK,)

    def body(a0_ref, a1_ref, a2_ref, a3_ref, w_ref, b_ref, ni_ref, mns_ref,
             we_ref, wd_ref, enc_ref, d_ref):
        a_refs = (a0_ref, a1_ref, a2_ref, a3_ref)
        acc = jnp.dot(a0_ref[...], w_ref[pl.ds(0, 128), :],
                      preferred_element_type=jnp.float32)
        for cc in range(1, 4):
            acc += jnp.dot(a_refs[cc][...], w_ref[pl.ds(cc * 128, 128), :],
                           preferred_element_type=jnp.float32)
        enc = jnp.maximum(acc * ni_ref[...] + b_ref[...], 0.0)
        enc_ref[...] = enc
        wed = jnp.dot(we_ref[...], wd_ref[...],
                      preferred_element_type=jnp.float32)
        d_ref[...] = jnp.dot(enc * mns_ref[...], wed,
                             preferred_element_type=jnp.float32)

    mspec = pl.BlockSpec((MBK, 128), lambda i: (i, 0))
    return pl.pallas_call(
        body,
        grid=grid,
        in_specs=[
            mspec, mspec, mspec, mspec,
            pl.BlockSpec((512, 512), lambda i: (0, 0)),
            pl.BlockSpec((1, 512), lambda i: (0, 0)),
            pl.BlockSpec((MBK, 1), lambda i: (i, 0)),
            pl.BlockSpec((MBK, 1), lambda i: (i, 0)),
            pl.BlockSpec((512, 512), lambda i: (0, 0)),
            pl.BlockSpec((512, 128), lambda i: (0, 0)),
        ],
        out_specs=[
            pl.BlockSpec((MBK, 512), lambda i: (i, 0)),
            pl.BlockSpec((MBK, 128), lambda i: (i, 0)),
        ],
        out_shape=[
            jax.ShapeDtypeStruct((NP, 512), jnp.float32),
            jax.ShapeDtypeStruct((NP, 128), jnp.float32),
        ],
    )(aggc[0], aggc[1], aggc[2], aggc[3], W2, b2, normin, mns, W_e2d, W_dec)


def _tc_final(agg3, b_dec, normin, maskcol, x_pad, NP, MBK):
    grid = (NP // MBK,)

    def body(a_ref, b_ref, ni_ref, m_ref, x_ref, rec_ref, loss_ref):
        i = pl.program_id(0)
        r = (a_ref[0] + a_ref[1]) * ni_ref[...] + b_ref[...]
        rec_ref[...] = r
        w = 1.0 - m_ref[...]
        x = x_ref[...]
        rnorm = jnp.sqrt(jnp.sum(r * r, axis=-1, keepdims=True))
        xnorm = jnp.sqrt(jnp.sum(x * x, axis=-1, keepdims=True))
        rn = r / jnp.maximum(rnorm, 1e-12)
        xn = x / jnp.maximum(xnorm, 1e-12)
        cos = jnp.sum(rn * xn, axis=-1, keepdims=True)
        contrib = jnp.sum(w * (1.0 - cos) ** 2, keepdims=True).reshape(1, 1)

        @pl.when(i == 0)
        def _():
            loss_ref[...] = contrib

        @pl.when(i > 0)
        def _():
            loss_ref[...] += contrib

    return pl.pallas_call(
        body,
        grid=grid,
        in_specs=[
            pl.BlockSpec((2, MBK, 128), lambda i: (0, i, 0)),
            pl.BlockSpec((1, 128), lambda i: (0, 0)),
            pl.BlockSpec((MBK, 1), lambda i: (i, 0)),
            pl.BlockSpec((MBK, 1), lambda i: (i, 0)),
            pl.BlockSpec((MBK, 128), lambda i: (i, 0)),
        ],
        out_specs=[
            pl.BlockSpec((MBK, 128), lambda i: (i, 0)),
            pl.BlockSpec((1, 1), lambda i: (0, 0)),
        ],
        out_shape=[
            jax.ShapeDtypeStruct((NP, 128), jnp.float32),
            jax.ShapeDtypeStruct((1, 1), jnp.float32),
        ],
    )(agg3, b_dec, normin, maskcol, x_pad)


def kernel(x, edge_index, mask_nodes, enc_mask_token,
           W_enc1, b_enc1, W_enc2, b_enc2, W_e2d, W_dec, b_dec):
    N = x.shape[0]
    num_mask = mask_nodes.shape[0]
    NP = ((N + NS * 40 - 1) // (NS * 40)) * (NS * 40)
    MBK = NP // 8

    x_pad = jnp.pad(x, ((0, NP - N), (0, 0)))
    maskcol = jnp.ones((NP, 1), jnp.float32).at[mask_nodes].set(0.0)
    zpad = jnp.zeros((NP, 128), jnp.float32)
    ones128 = jnp.ones((MB, 128), jnp.float32)

    E = edge_index.shape[1]
    EDIV = MB * NS * NC * 2
    EP = ((E + EDIV - 1) // EDIV) * EDIV
    pad_nodes = N + jnp.arange(EP - E, dtype=jnp.int32) % (NP - N)
    edge_index = jnp.concatenate(
        [edge_index, jnp.stack([pad_nodes, pad_nodes])], axis=1)

    degp = _sc_degrees(edge_index, zpad, ones128, NP)
    oxn, normin, normout, mns = _tc_prep(
        x_pad, degp, maskcol, enc_mask_token, NP, MBK)

    agg1 = _sc_agg128(oxn, edge_index, zpad, NP)
    h1n = _tc_layer1(agg1, W_enc1, b_enc1.reshape(1, -1), normin, normout,
                     NP, MBK)
    agg2c = _sc_agg512(h1n[0], h1n[1], h1n[2], h1n[3], edge_index, zpad, NP)
    enc_pad, d = _tc_layer2(agg2c, W_enc2, b_enc2.reshape(1, -1), normin, mns,
                            W_e2d, W_dec, NP, MBK)
    agg3 = _sc_agg128(d, edge_index, zpad, NP)
    recon_pad, loss_acc = _tc_final(agg3, b_dec.reshape(1, -1), normin,
                                    maskcol, x_pad, NP, MBK)

    enc_rep = enc_pad[:N]
    recon = recon_pad[:N]
    loss = (loss_acc[0, 0] / num_mask).astype(jnp.float32)
    return (enc_rep, recon, loss)


# speedup vs baseline: 2.3880x; 1.0010x over previous
import functools

import jax
import jax.numpy as jnp
from jax import lax
from jax.experimental import pallas as pl
from jax.experimental.pallas import tpu as pltpu
from jax.experimental.pallas import tpu_sc as plsc

NC = 2
NS = 16
MB = 128

def _mesh():
    return plsc.VectorSubcoreMesh(core_axis_name="c", subcore_axis_name="s",
                                  num_cores=NC, num_subcores=NS)


def _sc_degrees(edge_index, zpad, ones128, NP):
    E = edge_index.shape[1]
    nmb = E // MB
    nmb_core = nmb // NC
    rows_t = NP // NS
    nit = nmb_core // NS

    nit_all = nmb // NS

    @functools.partial(
        pl.kernel,
        out_type=jax.ShapeDtypeStruct((2, NP, 128), jnp.float32),
        mesh=_mesh(),
        scratch_types=[
            pltpu.VMEM((2, MB), jnp.int32),
            pltpu.VMEM((MB, 128), jnp.float32),
            pltpu.VMEM_SHARED((NP, 128), jnp.float32),
            pltpu.SemaphoreType.DMA,
            pltpu.SemaphoreType.DMA,
        ],
    )
    def deg_kernel(ei, zp, ones_hbm, out, idx2, ones_v, acc, isem0, isem1):
        c = lax.axis_index("c")
        s = lax.axis_index("s")
        r0 = s * rows_t
        pltpu.sync_copy(ones_hbm, ones_v)
        pltpu.sync_copy(zp.at[pl.ds(r0, rows_t), :],
                        acc.at[pl.ds(r0, rows_t), :])
        plsc.subcore_barrier()

        def run_pass(which):
            def start_idx(i, buf, isem):
                base = (i * NS + s) * MB
                pltpu.async_copy(ei.at[which, pl.ds(base, MB)],
                                 idx2.at[buf], isem)

            def wait_idx(i, buf, isem):
                base = (i * NS + s) * MB
                pltpu.make_async_copy(ei.at[which, pl.ds(base, MB)],
                                      idx2.at[buf], isem).wait()

            start_idx(0, 0, isem0)
            start_idx(1, 1, isem1)

            def body(g, _):
                i = 2 * g
                wait_idx(i, 0, isem0)
                pltpu.sync_copy(ones_v, acc.at[idx2.at[0]], add=True)

                @pl.when(i + 2 < nit_all)
                def _():
                    start_idx(i + 2, 0, isem0)

                wait_idx(i + 1, 1, isem1)
                pltpu.sync_copy(ones_v, acc.at[idx2.at[1]], add=True)

                @pl.when(i + 3 < nit_all)
                def _():
                    start_idx(i + 3, 1, isem1)

                return 0

            lax.fori_loop(0, nit_all // 2, body, 0)
            plsc.subcore_barrier()
            pltpu.sync_copy(acc.at[pl.ds(r0, rows_t), :],
                            out.at[which, pl.ds(r0, rows_t), :])

        @pl.when(c == 0)
        def _():
            run_pass(0)

        @pl.when(c == 1)
        def _():
            run_pass(1)

    return deg_kernel(edge_index, zpad, ones128)


def _agg_edge_loop(tab, ei, acc, sidx2, didx2, rows0, rows1,
                   gsem0, gsem1, isem0, isem1, mb0_of, nit):
    def start_idx(i, buf, isem):
        base = mb0_of(i) * MB
        pltpu.async_copy(ei.at[0, pl.ds(base, MB)], sidx2.at[buf], isem)
        pltpu.async_copy(ei.at[1, pl.ds(base, MB)], didx2.at[buf], isem)

    def wait_idx(i, buf, isem):
        base = mb0_of(i) * MB
        pltpu.make_async_copy(ei.at[0, pl.ds(base, MB)], sidx2.at[buf],
                              isem).wait()
        pltpu.make_async_copy(ei.at[1, pl.ds(base, MB)], didx2.at[buf],
                              isem).wait()

    def start_gather(buf_idx, rows, gsem):
        pltpu.async_copy(tab.at[sidx2.at[buf_idx]], rows, gsem)

    def wait_gather(buf_idx, rows, gsem):
        pltpu.make_async_copy(tab.at[sidx2.at[buf_idx]], rows, gsem).wait()

    start_idx(0, 0, isem0)
    start_idx(1, 1, isem1)
    wait_idx(0, 0, isem0)
    start_gather(0, rows0, gsem0)

    def body(g, _):
        i = 2 * g
        wait_idx(i + 1, 1, isem1)
        start_gather(1, rows1, gsem1)
        wait_gather(0, rows0, gsem0)
        pltpu.sync_copy(rows0, acc.at[didx2.at[0]], add=True)

        @pl.when(i + 2 < nit)
        def _():
            start_idx(i + 2, 0, isem0)
            wait_idx(i + 2, 0, isem0)
            start_gather(0, rows0, gsem0)

        wait_gather(1, rows1, gsem1)
        pltpu.sync_copy(rows1, acc.at[didx2.at[1]], add=True)

        @pl.when(i + 3 < nit)
        def _():
            start_idx(i + 3, 1, isem1)

        return 0

    lax.fori_loop(0, nit // 2, body, 0)


def _sc_agg128(table, edge_index, zpad, NP):
    E = edge_index.shape[1]
    nmb = E // MB
    nmb_core = nmb // NC
    rows_t = NP // NS
    nit = nmb_core // NS

    @functools.partial(
        pl.kernel,
        out_type=jax.ShapeDtypeStruct((NC, NP, 128), jnp.float32),
        mesh=_mesh(),
        scratch_types=[
            pltpu.VMEM((2, MB), jnp.int32),
            pltpu.VMEM((2, MB), jnp.int32),
            pltpu.VMEM((MB, 128), jnp.float32),
            pltpu.VMEM((MB, 128), jnp.float32),
            pltpu.VMEM_SHARED((NP, 128), jnp.float32),
            pltpu.SemaphoreType.DMA,
            pltpu.SemaphoreType.DMA,
            pltpu.SemaphoreType.DMA,
            pltpu.SemaphoreType.DMA,
        ],
    )
    def agg_kernel(tab, ei, zp, out, sidx2, didx2, rows0, rows1, acc,
                   gsem0, gsem1, isem0, isem1):
        c = lax.axis_index("c")
        s = lax.axis_index("s")
        r0 = s * rows_t
        pltpu.sync_copy(zp.at[pl.ds(r0, rows_t), :],
                        acc.at[pl.ds(r0, rows_t), :])
        plsc.subcore_barrier()
        _agg_edge_loop(tab, ei, acc, sidx2, didx2, rows0, rows1,
                       gsem0, gsem1, isem0, isem1,
                       lambda i: c * nmb_core + i * NS + s, nit)
        plsc.subcore_barrier()
        pltpu.sync_copy(acc.at[pl.ds(r0, rows_t), :],
                        out.at[c, pl.ds(r0, rows_t), :])

    return agg_kernel(table, edge_index, zpad)


def _sc_agg512(t0, t1, t2, t3, edge_index, zpad, NP):
    E = edge_index.shape[1]
    nmb = E // MB
    rows_t = NP // NS
    ot = jax.ShapeDtypeStruct((NP, 128), jnp.float32)

    nit = nmb // NS

    @functools.partial(
        pl.kernel,
        out_type=(ot, ot, ot, ot),
        mesh=_mesh(),
        scratch_types=[
            pltpu.VMEM((2, MB), jnp.int32),
            pltpu.VMEM((2, MB), jnp.int32),
            pltpu.VMEM((MB, 128), jnp.float32),
            pltpu.VMEM((MB, 128), jnp.float32),
            pltpu.VMEM_SHARED((NP, 128), jnp.float32),
            pltpu.SemaphoreType.DMA,
            pltpu.SemaphoreType.DMA,
            pltpu.SemaphoreType.DMA,
            pltpu.SemaphoreType.DMA,
        ],
    )
    def agg_kernel(a0, a1, a2, a3, ei, zp, o0, o1, o2, o3,
                   sidx2, didx2, rows0, rows1, acc,
                   gsem0, gsem1, isem0, isem1):
        c = lax.axis_index("c")
        s = lax.axis_index("s")
        r0 = s * rows_t

        def run_chunk(tab, out):
            pltpu.sync_copy(zp.at[pl.ds(r0, rows_t), :],
                            acc.at[pl.ds(r0, rows_t), :])
            plsc.subcore_barrier()
            _agg_edge_loop(tab, ei, acc, sidx2, didx2, rows0, rows1,
                           gsem0, gsem1, isem0, isem1,
                           lambda i: i * NS + s, nit)
            plsc.subcore_barrier()
            pltpu.sync_copy(acc.at[pl.ds(r0, rows_t), :],
                            out.at[pl.ds(r0, rows_t), :])

        @pl.when(c == 0)
        def _():
            run_chunk(a0, o0)
            plsc.subcore_barrier()
            run_chunk(a1, o1)

        @pl.when(c == 1)
        def _():
            run_chunk(a2, o2)
            plsc.subcore_barrier()
            run_chunk(a3, o3)

    return agg_kernel(t0, t1, t2, t3, edge_index, zpad)


def _tc_prep(x_pad, degp, maskcol, token, NP, MBK):
    grid = NP // MBK

    def body(x_ref, deg_ref, m_ref, tok_ref, oxn_ref, ni_ref, no_ref, mns_ref):
        dego = deg_ref[0, :, 0:1]
        degi = deg_ref[1, :, 0:1]
        no = jnp.where(dego > 0, lax.rsqrt(jnp.maximum(dego, 1e-30)), 0.0)
        ni = jnp.where(degi > 0, lax.rsqrt(jnp.maximum(degi, 1e-30)), 0.0)
        m = m_ref[...]
        ox = x_ref[...] * m + (1.0 - m) * tok_ref[...]
        oxn_ref[...] = ox * no
        ni_ref[...] = ni
        no_ref[...] = no
        mns_ref[...] = m * no

    return pl.pallas_call(
        body,
        grid=(grid,),
        in_specs=[
            pl.BlockSpec((MBK, 128), lambda i: (i, 0)),
            pl.BlockSpec((2, MBK, 128), lambda i: (0, i, 0)),
            pl.BlockSpec((MBK, 1), lambda i: (i, 0)),
            pl.BlockSpec((1, 128), lambda i: (0, 0)),
        ],
        out_specs=[
            pl.BlockSpec((MBK, 128), lambda i: (i, 0)),
            pl.BlockSpec((MBK, 1), lambda i: (i, 0)),
            pl.BlockSpec((MBK, 1), lambda i: (i, 0)),
            pl.BlockSpec((MBK, 1), lambda i: (i, 0)),
        ],
        out_shape=[
            jax.ShapeDtypeStruct((NP, 128), jnp.float32),
            jax.ShapeDtypeStruct((NP, 1), jnp.float32),
            jax.ShapeDtypeStruct((NP, 1), jnp.float32),
            jax.ShapeDtypeStruct((NP, 1), jnp.float32),
        ],
    )(x_pad, degp, maskcol, token)


def _tc_layer1(agg1, W1, b1, normin, normout, NP, MBK):
    grid = (NP // MBK, 4)

    def body(a_ref, w_ref, b_ref, ni_ref, no_ref, o_ref):
        a = a_ref[0] + a_ref[1]
        acc = jnp.dot(a, w_ref[...], preferred_element_type=jnp.float32)
        h = jnp.maximum(acc * ni_ref[...] + b_ref[...], 0.0)
        o_ref[0] = h * no_ref[...]

    return pl.pallas_call(
        body,
        grid=grid,
        in_specs=[
            pl.BlockSpec((2, MBK, 128), lambda i, c: (0, i, 0)),
            pl.BlockSpec((128, 128), lambda i, c: (0, c)),
            pl.BlockSpec((1, 128), lambda i, c: (0, c)),
            pl.BlockSpec((MBK, 1), lambda i, c: (i, 0)),
            pl.BlockSpec((MBK, 1), lambda i, c: (i, 0)),
        ],
        out_specs=pl.BlockSpec((1, MBK, 128), lambda i, c: (c, i, 0)),
        out_shape=jax.ShapeDtypeStruct((4, NP, 128), jnp.float32),
    )(agg1, W1, b1, normin, normout)


def _tc_layer2(aggc, W2, b2, normin, mns, W_e2d, W_dec, NP, MBK):
    grid = (NP // MBK,)

    def body(a0_ref, a1_ref, a2_ref, a3_ref, w_ref, b_ref, ni_ref, mns_ref,
             we_ref, wd_ref, enc_ref, d_ref):
        a_refs = (a0_ref, a1_ref, a2_ref, a3_ref)
        acc = jnp.dot(a0_ref[...], w_ref[pl.ds(0, 128), :],
                      preferred_element_type=jnp.float32)
        for cc in range(1, 4):
            acc += jnp.dot(a_refs[cc][...], w_ref[pl.ds(cc * 128, 128), :],
                           preferred_element_type=jnp.float32)
        enc = jnp.maximum(acc * ni_ref[...] + b_ref[...], 0.0)
        enc_ref[...] = enc
        wed = jnp.dot(we_ref[...], wd_ref[...],
                      preferred_element_type=jnp.float32)
        d_ref[...] = jnp.dot(enc * mns_ref[...], wed,
                             preferred_element_type=jnp.float32)

    mspec = pl.BlockSpec((MBK, 128), lambda i: (i, 0))
    return pl.pallas_call(
        body,
        grid=grid,
        in_specs=[
            mspec, mspec, mspec, mspec,
            pl.BlockSpec((512, 512), lambda i: (0, 0)),
            pl.BlockSpec((1, 512), lambda i: (0, 0)),
            pl.BlockSpec((MBK, 1), lambda i: (i, 0)),
            pl.BlockSpec((MBK, 1), lambda i: (i, 0)),
            pl.BlockSpec((512, 512), lambda i: (0, 0)),
            pl.BlockSpec((512, 128), lambda i: (0, 0)),
        ],
        out_specs=[
            pl.BlockSpec((MBK, 512), lambda i: (i, 0)),
            pl.BlockSpec((MBK, 128), lambda i: (i, 0)),
        ],
        out_shape=[
            jax.ShapeDtypeStruct((NP, 512), jnp.float32),
            jax.ShapeDtypeStruct((NP, 128), jnp.float32),
        ],
    )(aggc[0], aggc[1], aggc[2], aggc[3], W2, b2, normin, mns, W_e2d, W_dec)


def _tc_final(agg3, b_dec, normin, maskcol, x_pad, NP, MBK):
    grid = (NP // MBK,)

    def body(a_ref, b_ref, ni_ref, m_ref, x_ref, rec_ref, loss_ref):
        i = pl.program_id(0)
        r = (a_ref[0] + a_ref[1]) * ni_ref[...] + b_ref[...]
        rec_ref[...] = r
        w = 1.0 - m_ref[...]
        x = x_ref[...]
        rnorm = jnp.sqrt(jnp.sum(r * r, axis=-1, keepdims=True))
        xnorm = jnp.sqrt(jnp.sum(x * x, axis=-1, keepdims=True))
        rn = r / jnp.maximum(rnorm, 1e-12)
        xn = x / jnp.maximum(xnorm, 1e-12)
        cos = jnp.sum(rn * xn, axis=-1, keepdims=True)
        contrib = jnp.sum(w * (1.0 - cos) ** 2, keepdims=True).reshape(1, 1)

        @pl.when(i == 0)
        def _():
            loss_ref[...] = contrib

        @pl.when(i > 0)
        def _():
            loss_ref[...] += contrib

    return pl.pallas_call(
        body,
        grid=grid,
        in_specs=[
            pl.BlockSpec((2, MBK, 128), lambda i: (0, i, 0)),
            pl.BlockSpec((1, 128), lambda i: (0, 0)),
            pl.BlockSpec((MBK, 1), lambda i: (i, 0)),
            pl.BlockSpec((MBK, 1), lambda i: (i, 0)),
            pl.BlockSpec((MBK, 128), lambda i: (i, 0)),
        ],
        out_specs=[
            pl.BlockSpec((MBK, 128), lambda i: (i, 0)),
            pl.BlockSpec((1, 1), lambda i: (0, 0)),
        ],
        out_shape=[
            jax.ShapeDtypeStruct((NP, 128), jnp.float32),
            jax.ShapeDtypeStruct((1, 1), jnp.float32),
        ],
    )(agg3, b_dec, normin, maskcol, x_pad)


def kernel(x, edge_index, mask_nodes, enc_mask_token,
           W_enc1, b_enc1, W_enc2, b_enc2, W_e2d, W_dec, b_dec):
    N = x.shape[0]
    num_mask = mask_nodes.shape[0]
    NP = ((N + NS * 40 - 1) // (NS * 40)) * (NS * 40)
    MBK = NP // 8

    x_pad = jnp.pad(x, ((0, NP - N), (0, 0)))
    maskcol = jnp.ones((NP, 1), jnp.float32).at[mask_nodes].set(0.0)
    zpad = jnp.zeros((NP, 128), jnp.float32)
    ones128 = jnp.ones((MB, 128), jnp.float32)

    E = edge_index.shape[1]
    EDIV = MB * NS * NC * 2
    EP = ((E + EDIV - 1) // EDIV) * EDIV
    pad_nodes = N + jnp.arange(EP - E, dtype=jnp.int32) % (NP - N)
    edge_index = jnp.concatenate(
        [edge_index, jnp.stack([pad_nodes, pad_nodes])], axis=1)

    degp = _sc_degrees(edge_index, zpad, ones128, NP)
    oxn, normin, normout, mns = _tc_prep(
        x_pad, degp, maskcol, enc_mask_token, NP, MBK)

    agg1 = _sc_agg128(oxn, edge_index, zpad, NP)
    h1n = _tc_layer1(agg1, W_enc1, b_enc1.reshape(1, -1), normin, normout,
                     NP, MBK)
    agg2c = _sc_agg512(h1n[0], h1n[1], h1n[2], h1n[3], edge_index, zpad, NP)
    enc_pad, d = _tc_layer2(agg2c, W_enc2, b_enc2.reshape(1, -1), normin, mns,
                            W_e2d, W_dec, NP, MBK)
    agg3 = _sc_agg128(d, edge_index, zpad, NP)
    recon_pad, loss_acc = _tc_final(agg3, b_dec.reshape(1, -1), normin,
                                    maskcol, x_pad, NP, MBK)

    enc_rep = enc_pad[:N]
    recon = recon_pad[:N]
    loss = (loss_acc[0, 0] / num_mask).astype(jnp.float32)
    return (enc_rep, recon, loss)
